# Initial kernel scaffold; baseline (speedup 1.0000x reference)
import jax
import jax.numpy as jnp
from jax.experimental import pallas as pl


def kernel(x, edge_index, edge_weight, W_l1, b_l1, W_r1, b_r1, W_l2, b_l2, W_r2, b_r2):
    raise NotImplementedError("write your pallas kernel here")


# speedup vs baseline: 7.8786x; 7.8786x over previous
import functools

import jax
import jax.numpy as jnp
from jax import lax
from jax.experimental import pallas as pl
from jax.experimental.pallas import tpu as pltpu
from jax.experimental.pallas import tpu_sc as plsc

N_NODES = 10000
N_EDGES = 320000
D_IN = 128
D_HID = 16
D_OUT = 128

NC = 2
NS = 16
NW = NC * NS
GROUP = 128
GPT = 80
NE_PAD = NW * GPT * GROUP
N_PAD = 10240
ROWS_PER_SUB = N_PAD // NS
M_BLK = 1000


def _sc_pass(feat_hbm, src_hbm, dst_hbm, w_hbm, val_hbm, z16_hbm, z1_hbm,
             acc_out, cnt_out, src_v, dst_v, w_v, val_v, rows_v,
             acc_sh, cnt_sh, sem, *, do_cnt):
    c = lax.axis_index("c")
    s = lax.axis_index("s")
    wid = c * NS + s

    pltpu.sync_copy(z16_hbm.at[pl.ds(s * ROWS_PER_SUB, ROWS_PER_SUB)],
                    acc_sh.at[pl.ds(s * ROWS_PER_SUB, ROWS_PER_SUB)])
    if do_cnt:
        @pl.when(s == 0)
        def _():
            pltpu.sync_copy(z1_hbm, cnt_sh)
    plsc.subcore_barrier()

    base = wid * GPT
    pltpu.sync_copy(src_hbm.at[pl.ds(base, GPT)], src_v)
    pltpu.sync_copy(dst_hbm.at[pl.ds(base, GPT)], dst_v)
    pltpu.sync_copy(w_hbm.at[pl.ds(base, GPT)], w_v)
    if do_cnt:
        pltpu.sync_copy(val_hbm.at[pl.ds(base, GPT)], val_v)

    iota = lax.iota(jnp.int32, 16)

    def group_body(j, _):
        pltpu.async_copy(feat_hbm.at[src_v.at[j]], rows_v, sem).wait()

        def edge_body(i, _):
            ii = jnp.full((16,), i, dtype=jnp.int32)
            jj = jnp.full((16,), j, dtype=jnp.int32)
            wsplat = plsc.load_gather(w_v, [jj, ii])
            row = plsc.load_gather(rows_v, [ii, iota])
            plsc.store_scatter(rows_v, [ii, iota], row * wsplat)
            return 0

        lax.fori_loop(0, GROUP, edge_body, 0)
        pltpu.sync_copy(rows_v, acc_sh.at[dst_v.at[j]], add=True)
        if do_cnt:
            pltpu.sync_copy(val_v.at[j], cnt_sh.at[dst_v.at[j]], add=True)
        return 0

    lax.fori_loop(0, GPT, group_body, 0)
    plsc.subcore_barrier()

    pltpu.sync_copy(acc_sh.at[pl.ds(s * ROWS_PER_SUB, ROWS_PER_SUB)],
                    acc_out.at[c, pl.ds(s * ROWS_PER_SUB, ROWS_PER_SUB)])
    if do_cnt:
        @pl.when(s == 0)
        def _():
            pltpu.sync_copy(cnt_sh, cnt_out.at[pl.ds(c * N_PAD, N_PAD)])


def _make_sc_pass(do_cnt):
    mesh = plsc.VectorSubcoreMesh(core_axis_name="c", subcore_axis_name="s",
                                  num_cores=NC, num_subcores=NS)
    out_type = (jax.ShapeDtypeStruct((NC, N_PAD, D_HID), jnp.float32),
                jax.ShapeDtypeStruct((NC * N_PAD,), jnp.float32))
    scratch = (
        pltpu.VMEM((GPT, GROUP), jnp.int32),
        pltpu.VMEM((GPT, GROUP), jnp.int32),
        pltpu.VMEM((GPT, GROUP), jnp.float32),
        pltpu.VMEM((GPT, GROUP), jnp.float32),
        pltpu.VMEM((GROUP, D_HID), jnp.float32),
        pltpu.VMEM_SHARED((N_PAD, D_HID), jnp.float32),
        pltpu.VMEM_SHARED((N_PAD,), jnp.float32),
        pltpu.SemaphoreType.DMA,
    )

    def body(feat, src, dst, w, val, z16, z1, acc_out, cnt_out,
             src_v, dst_v, w_v, val_v, rows_v, acc_sh, cnt_sh, sem):
        _sc_pass(feat, src, dst, w, val, z16, z1, acc_out, cnt_out,
                 src_v, dst_v, w_v, val_v, rows_v, acc_sh, cnt_sh, sem,
                 do_cnt=do_cnt)

    return pl.kernel(body, out_type=out_type, mesh=mesh,
                     scratch_types=scratch,
                     compiler_params=pltpu.CompilerParams(
                         needs_layout_passes=False,
                         use_tc_tiling_on_sc=False))


_sc_pass1 = _make_sc_pass(True)
_sc_pass2 = _make_sc_pass(False)


def _mm_body(x_ref, wl_ref, wr_ref, xl_ref, xr_ref):
    x = x_ref[...]
    xl_ref[...] = jnp.dot(x, wl_ref[...], preferred_element_type=jnp.float32)
    xr_ref[...] = jnp.dot(x, wr_ref[...], preferred_element_type=jnp.float32)


def _h_body(p0_ref, p1_ref, c0_ref, c1_ref, xr_ref, b_ref, h_ref):
    cnt = jnp.maximum(c0_ref[...] + c1_ref[...], 1.0)
    mean = (p0_ref[...] + p1_ref[...]) / cnt
    h_ref[...] = jnp.maximum(mean + xr_ref[...] + b_ref[...], 0.0)


def _out_body(q0_ref, q1_ref, c0_ref, c1_ref, h_ref, wl_ref, wr_ref, b_ref,
              o_ref):
    cnt = jnp.maximum(c0_ref[...] + c1_ref[...], 1.0)
    mean = (q0_ref[...] + q1_ref[...]) / cnt
    o_ref[...] = (jnp.dot(mean, wl_ref[...], preferred_element_type=jnp.float32)
                  + jnp.dot(h_ref[...], wr_ref[...],
                            preferred_element_type=jnp.float32)
                  + b_ref[...])


def _row_spec(width):
    return pl.BlockSpec((M_BLK, width), lambda i: (i, 0))


def _full_spec(shape):
    return pl.BlockSpec(shape, lambda i: (0,) * len(shape))


def kernel(x, edge_index, edge_weight, W_l1, b_l1, W_r1, b_r1,
           W_l2, b_l2, W_r2, b_r2):
    f32 = jnp.float32
    pad = NE_PAD - N_EDGES
    src = jnp.concatenate(
        [edge_index[0].astype(jnp.int32), jnp.zeros((pad,), jnp.int32)]
    ).reshape(NW * GPT, GROUP)
    dst = jnp.concatenate(
        [edge_index[1].astype(jnp.int32), jnp.zeros((pad,), jnp.int32)]
    ).reshape(NW * GPT, GROUP)
    w = jnp.concatenate([edge_weight, jnp.zeros((pad,), f32)]
                        ).reshape(NW * GPT, GROUP)
    val = jnp.concatenate([jnp.ones((N_EDGES,), f32), jnp.zeros((pad,), f32)]
                          ).reshape(NW * GPT, GROUP)
    z16 = jnp.zeros((N_PAD, D_HID), f32)
    z1 = jnp.zeros((N_PAD,), f32)

    grid = N_NODES // M_BLK

    xl, xr = pl.pallas_call(
        _mm_body,
        grid=(grid,),
        in_specs=[_row_spec(D_IN), _full_spec((D_IN, D_HID)),
                  _full_spec((D_IN, D_HID))],
        out_specs=[_row_spec(D_HID), _row_spec(D_HID)],
        out_shape=[jax.ShapeDtypeStruct((N_NODES, D_HID), f32)] * 2,
    )(x, W_l1, W_r1)

    acc1, cnt = _sc_pass1(xl, src, dst, w, val, z16, z1)
    cnt = cnt.reshape(NC, N_PAD)
    c0 = cnt[0, :N_NODES].reshape(N_NODES, 1)
    c1 = cnt[1, :N_NODES].reshape(N_NODES, 1)
    p0 = acc1[0, :N_NODES]
    p1 = acc1[1, :N_NODES]

    b1 = (b_l1 + b_r1).reshape(1, D_HID)
    h = pl.pallas_call(
        _h_body,
        grid=(grid,),
        in_specs=[_row_spec(D_HID), _row_spec(D_HID), _row_spec(1),
                  _row_spec(1), _row_spec(D_HID), _full_spec((1, D_HID))],
        out_specs=_row_spec(D_HID),
        out_shape=jax.ShapeDtypeStruct((N_NODES, D_HID), f32),
    )(p0, p1, c0, c1, xr, b1)

    acc2, _ = _sc_pass2(h, src, dst, w, val, z16, z1)
    q0 = acc2[0, :N_NODES]
    q1 = acc2[1, :N_NODES]

    b2 = (b_l2 + b_r2).reshape(1, D_OUT)
    out = pl.pallas_call(
        _out_body,
        grid=(grid,),
        in_specs=[_row_spec(D_HID), _row_spec(D_HID), _row_spec(1),
                  _row_spec(1), _row_spec(D_HID), _full_spec((D_HID, D_OUT)),
                  _full_spec((D_HID, D_OUT)), _full_spec((1, D_OUT))],
        out_specs=_row_spec(D_OUT),
        out_shape=jax.ShapeDtypeStruct((N_NODES, D_OUT), f32),
    )(q0, q1, c0, c1, h, W_l2, W_r2, b2)

    return out


# speedup vs baseline: 11.3223x; 1.4371x over previous
import functools

import jax
import jax.numpy as jnp
from jax import lax
from jax.experimental import pallas as pl
from jax.experimental.pallas import tpu as pltpu
from jax.experimental.pallas import tpu_sc as plsc

N_NODES = 10000
N_EDGES = 320000
D_IN = 128
D_HID = 16
D_OUT = 128

NC = 2
NS = 16
NW = NC * NS
GROUP = 128
GPT = 80
NE_PAD = NW * GPT * GROUP
N_PAD = 10240
ROWS_PER_SUB = N_PAD // NS
M_BLK = 1000


def _sc_pass(feat_hbm, src_hbm, dst_hbm, w_hbm, val_hbm, z16_hbm, z1_hbm,
             acc_out, cnt_out, src_v, dst_v, w_v, val_v, rows0, rows1,
             acc_sh, cnt_sh, gsem0, gsem1, *, do_cnt):
    c = lax.axis_index("c")
    s = lax.axis_index("s")
    wid = c * NS + s
    rows = (rows0, rows1)
    gsem = (gsem0, gsem1)

    pltpu.sync_copy(z16_hbm.at[pl.ds(s * ROWS_PER_SUB, ROWS_PER_SUB)],
                    acc_sh.at[pl.ds(s * ROWS_PER_SUB, ROWS_PER_SUB)])
    if do_cnt:
        @pl.when(s == 0)
        def _():
            pltpu.sync_copy(z1_hbm, cnt_sh)

    base = wid * GPT
    pltpu.sync_copy(src_hbm.at[pl.ds(base, GPT)], src_v)
    pltpu.sync_copy(dst_hbm.at[pl.ds(base, GPT)], dst_v)
    pltpu.sync_copy(w_hbm.at[pl.ds(base, GPT)], w_v)
    if do_cnt:
        pltpu.sync_copy(val_hbm.at[pl.ds(base, GPT)], val_v)
    plsc.subcore_barrier()

    iota = lax.iota(jnp.int32, 16)

    def gather_start(j, b):
        pltpu.async_copy(feat_hbm.at[src_v.at[j]], rows[b], gsem[b])

    gather_start(0, 0)

    def handle_group(j, b, prefetch):
        pltpu.make_async_copy(feat_hbm.at[src_v.at[j]], rows[b],
                              gsem[b]).wait()
        if prefetch is not None:
            if prefetch == "maybe":
                @pl.when(j + 1 < GPT)
                def _():
                    gather_start(j + 1, 1 - b)
            else:
                gather_start(j + 1, 1 - b)

        rv = rows[b]
        jj = jnp.full((16,), j, dtype=jnp.int32)

        @plsc.parallel_loop(0, GROUP, 1, unroll=8)
        def _(i):
            ii = jnp.full((16,), i, dtype=jnp.int32)
            wsplat = plsc.load_gather(w_v, [jj, ii])
            row = plsc.load_gather(rv, [ii, iota])
            plsc.store_scatter(rv, [ii, iota], row * wsplat)

        pltpu.sync_copy(rv, acc_sh.at[dst_v.at[j]], add=True)
        if do_cnt:
            pltpu.sync_copy(val_v.at[j], cnt_sh.at[dst_v.at[j]], add=True)

    def outer_body(j2, _):
        handle_group(2 * j2, 0, "always")
        handle_group(2 * j2 + 1, 1, "maybe")
        return 0

    lax.fori_loop(0, GPT // 2, outer_body, 0)
    plsc.subcore_barrier()

    pltpu.sync_copy(acc_sh.at[pl.ds(s * ROWS_PER_SUB, ROWS_PER_SUB)],
                    acc_out.at[c, pl.ds(s * ROWS_PER_SUB, ROWS_PER_SUB)])
    if do_cnt:
        @pl.when(s == 0)
        def _():
            pltpu.sync_copy(cnt_sh, cnt_out.at[pl.ds(c * N_PAD, N_PAD)])


def _make_sc_pass(do_cnt):
    mesh = plsc.VectorSubcoreMesh(core_axis_name="c", subcore_axis_name="s",
                                  num_cores=NC, num_subcores=NS)
    out_type = (jax.ShapeDtypeStruct((NC, N_PAD, D_HID), jnp.float32),
                jax.ShapeDtypeStruct((NC * N_PAD,), jnp.float32))
    scratch = (
        pltpu.VMEM((GPT, GROUP), jnp.int32),
        pltpu.VMEM((GPT, GROUP), jnp.int32),
        pltpu.VMEM((GPT, GROUP), jnp.float32),
        pltpu.VMEM((GPT, GROUP), jnp.float32),
        pltpu.VMEM((GROUP, D_HID), jnp.float32),
        pltpu.VMEM((GROUP, D_HID), jnp.float32),
        pltpu.VMEM_SHARED((N_PAD, D_HID), jnp.float32),
        pltpu.VMEM_SHARED((N_PAD,), jnp.float32),
        pltpu.SemaphoreType.DMA,
        pltpu.SemaphoreType.DMA,
    )

    def body(feat, src, dst, w, val, z16, z1, acc_out, cnt_out,
             src_v, dst_v, w_v, val_v, rows0, rows1, acc_sh, cnt_sh,
             gsem0, gsem1):
        _sc_pass(feat, src, dst, w, val, z16, z1, acc_out, cnt_out,
                 src_v, dst_v, w_v, val_v, rows0, rows1, acc_sh, cnt_sh,
                 gsem0, gsem1, do_cnt=do_cnt)

    return pl.kernel(body, out_type=out_type, mesh=mesh,
                     scratch_types=scratch,
                     compiler_params=pltpu.CompilerParams(
                         needs_layout_passes=False,
                         use_tc_tiling_on_sc=False))


_sc_pass1 = _make_sc_pass(True)
_sc_pass2 = _make_sc_pass(False)


def _mm_body(x_ref, wl_ref, wr_ref, xl_ref, xr_ref):
    x = x_ref[...]
    xl_ref[...] = jnp.dot(x, wl_ref[...], preferred_element_type=jnp.float32)
    xr_ref[...] = jnp.dot(x, wr_ref[...], preferred_element_type=jnp.float32)


def _h_body(p0_ref, p1_ref, c0_ref, c1_ref, xr_ref, b_ref, h_ref):
    cnt = jnp.maximum(c0_ref[...] + c1_ref[...], 1.0)
    mean = (p0_ref[...] + p1_ref[...]) / cnt
    h_ref[...] = jnp.maximum(mean + xr_ref[...] + b_ref[...], 0.0)


def _out_body(q0_ref, q1_ref, c0_ref, c1_ref, h_ref, wl_ref, wr_ref, b_ref,
              o_ref):
    cnt = jnp.maximum(c0_ref[...] + c1_ref[...], 1.0)
    mean = (q0_ref[...] + q1_ref[...]) / cnt
    o_ref[...] = (jnp.dot(mean, wl_ref[...], preferred_element_type=jnp.float32)
                  + jnp.dot(h_ref[...], wr_ref[...],
                            preferred_element_type=jnp.float32)
                  + b_ref[...])


def _row_spec(width):
    return pl.BlockSpec((M_BLK, width), lambda i: (i, 0))


def _full_spec(shape):
    return pl.BlockSpec(shape, lambda i: (0,) * len(shape))


def kernel(x, edge_index, edge_weight, W_l1, b_l1, W_r1, b_r1,
           W_l2, b_l2, W_r2, b_r2):
    f32 = jnp.float32
    pad = NE_PAD - N_EDGES
    src = jnp.concatenate(
        [edge_index[0].astype(jnp.int32), jnp.zeros((pad,), jnp.int32)]
    ).reshape(NW * GPT, GROUP)
    dst = jnp.concatenate(
        [edge_index[1].astype(jnp.int32), jnp.zeros((pad,), jnp.int32)]
    ).reshape(NW * GPT, GROUP)
    w = jnp.concatenate([edge_weight, jnp.zeros((pad,), f32)]
                        ).reshape(NW * GPT, GROUP)
    val = jnp.concatenate([jnp.ones((N_EDGES,), f32), jnp.zeros((pad,), f32)]
                          ).reshape(NW * GPT, GROUP)
    z16 = jnp.zeros((N_PAD, D_HID), f32)
    z1 = jnp.zeros((N_PAD,), f32)

    grid = N_NODES // M_BLK

    xl, xr = pl.pallas_call(
        _mm_body,
        grid=(grid,),
        in_specs=[_row_spec(D_IN), _full_spec((D_IN, D_HID)),
                  _full_spec((D_IN, D_HID))],
        out_specs=[_row_spec(D_HID), _row_spec(D_HID)],
        out_shape=[jax.ShapeDtypeStruct((N_NODES, D_HID), f32)] * 2,
    )(x, W_l1, W_r1)

    acc1, cnt = _sc_pass1(xl, src, dst, w, val, z16, z1)
    cnt = cnt.reshape(NC, N_PAD)
    c0 = cnt[0, :N_NODES].reshape(N_NODES, 1)
    c1 = cnt[1, :N_NODES].reshape(N_NODES, 1)
    p0 = acc1[0, :N_NODES]
    p1 = acc1[1, :N_NODES]

    b1 = (b_l1 + b_r1).reshape(1, D_HID)
    h = pl.pallas_call(
        _h_body,
        grid=(grid,),
        in_specs=[_row_spec(D_HID), _row_spec(D_HID), _row_spec(1),
                  _row_spec(1), _row_spec(D_HID), _full_spec((1, D_HID))],
        out_specs=_row_spec(D_HID),
        out_shape=jax.ShapeDtypeStruct((N_NODES, D_HID), f32),
    )(p0, p1, c0, c1, xr, b1)

    acc2, _ = _sc_pass2(h, src, dst, w, val, z16, z1)
    q0 = acc2[0, :N_NODES]
    q1 = acc2[1, :N_NODES]

    b2 = (b_l2 + b_r2).reshape(1, D_OUT)
    out = pl.pallas_call(
        _out_body,
        grid=(grid,),
        in_specs=[_row_spec(D_HID), _row_spec(D_HID), _row_spec(1),
                  _row_spec(1), _row_spec(D_HID), _full_spec((D_HID, D_OUT)),
                  _full_spec((D_HID, D_OUT)), _full_spec((1, D_OUT))],
        out_specs=_row_spec(D_OUT),
        out_shape=jax.ShapeDtypeStruct((N_NODES, D_OUT), f32),
    )(q0, q1, c0, c1, h, W_l2, W_r2, b2)

    return out


# speedup vs baseline: 13.6822x; 1.2084x over previous
import functools

import jax
import jax.numpy as jnp
from jax import lax
from jax.experimental import pallas as pl
from jax.experimental.pallas import tpu as pltpu
from jax.experimental.pallas import tpu_sc as plsc

N_NODES = 10000
N_EDGES = 320000
D_IN = 128
D_HID = 16
D_OUT = 128

NC = 2
NS = 16
NW = NC * NS
GROUP = 128
GPT = 80
NE_PAD = NW * GPT * GROUP
N_PAD = 10240
ROWS_PER_SUB = N_PAD // NS
M_BLK = 1000


NBUF = 4


def _sc_pass(feat_hbm, src_hbm, dst_hbm, w_hbm, val_hbm, z16_hbm, z1_hbm,
             acc_out, cnt_out, src_v, dst_v, w_v, val_v,
             rows0, rows1, rows2, rows3, acc_sh, cnt_sh,
             gsem0, gsem1, gsem2, gsem3, ssem0, ssem1, ssem2, ssem3, csem,
             *, do_cnt):
    c = lax.axis_index("c")
    s = lax.axis_index("s")
    wid = c * NS + s
    rows = (rows0, rows1, rows2, rows3)
    gsem = (gsem0, gsem1, gsem2, gsem3)
    ssem = (ssem0, ssem1, ssem2, ssem3)

    pltpu.sync_copy(z16_hbm.at[pl.ds(s * ROWS_PER_SUB, ROWS_PER_SUB)],
                    acc_sh.at[pl.ds(s * ROWS_PER_SUB, ROWS_PER_SUB)])
    if do_cnt:
        @pl.when(s == 0)
        def _():
            pltpu.sync_copy(z1_hbm, cnt_sh)

    base = wid * GPT
    pltpu.sync_copy(src_hbm.at[pl.ds(base, GPT)], src_v)
    pltpu.sync_copy(dst_hbm.at[pl.ds(base, GPT)], dst_v)
    pltpu.sync_copy(w_hbm.at[pl.ds(base, GPT)], w_v)
    if do_cnt:
        pltpu.sync_copy(val_hbm.at[pl.ds(base, GPT)], val_v)
    plsc.subcore_barrier()

    iota = lax.iota(jnp.int32, 16)

    def gather_start(j, b):
        pltpu.async_copy(feat_hbm.at[src_v.at[j]], rows[b], gsem[b])

    def gather_wait(j, b):
        pltpu.make_async_copy(feat_hbm.at[src_v.at[j]], rows[b],
                              gsem[b]).wait()

    def scatter_start(j, b):
        pltpu.async_copy(rows[b], acc_sh.at[dst_v.at[j]], ssem[b], add=True)

    def scatter_wait(j, b):
        pltpu.make_async_copy(rows[b], acc_sh.at[dst_v.at[j]],
                              ssem[b]).wait()

    def scale(j, b):
        rv = rows[b]
        jj = jnp.full((16,), j, dtype=jnp.int32)

        @plsc.parallel_loop(0, GROUP, 1, unroll=8)
        def _(i):
            ii = jnp.full((16,), i, dtype=jnp.int32)
            wsplat = plsc.load_gather(w_v, [jj, ii])
            row = plsc.load_gather(rv, [ii, iota])
            plsc.store_scatter(rv, [ii, iota], row * wsplat)

    gather_start(0, 0)
    gather_start(1, 1)

    def handle_group(j4, u):
        j = NBUF * j4 + u
        b = (u + 2) % NBUF
        gather_wait(j, u)
        if u < 2:
            @pl.when(j4 > 0)
            def _():
                scatter_wait(j - 2, b)
            gather_start(j + 2, b)
        else:
            @pl.when(j + 2 < GPT)
            def _():
                scatter_wait(j - 2, b)
                gather_start(j + 2, b)
        scale(j, u)
        scatter_start(j, u)
        if do_cnt:
            pltpu.async_copy(val_v.at[j], cnt_sh.at[dst_v.at[j]], csem,
                             add=True)

    def outer_body(j4, _):
        for u in range(NBUF):
            handle_group(j4, u)
        return 0

    lax.fori_loop(0, GPT // NBUF, outer_body, 0)
    for jt in range(GPT - NBUF, GPT):
        scatter_wait(jt, jt % NBUF)
    if do_cnt:
        def cnt_drain(j, _):
            pltpu.make_async_copy(val_v.at[j], cnt_sh.at[dst_v.at[j]],
                                  csem).wait()
            return 0
        lax.fori_loop(0, GPT, cnt_drain, 0)
    plsc.subcore_barrier()

    pltpu.sync_copy(acc_sh.at[pl.ds(s * ROWS_PER_SUB, ROWS_PER_SUB)],
                    acc_out.at[c, pl.ds(s * ROWS_PER_SUB, ROWS_PER_SUB)])
    if do_cnt:
        @pl.when(s == 0)
        def _():
            pltpu.sync_copy(cnt_sh, cnt_out.at[pl.ds(c * N_PAD, N_PAD)])


def _make_sc_pass(do_cnt):
    mesh = plsc.VectorSubcoreMesh(core_axis_name="c", subcore_axis_name="s",
                                  num_cores=NC, num_subcores=NS)
    out_type = (jax.ShapeDtypeStruct((NC, N_PAD, D_HID), jnp.float32),
                jax.ShapeDtypeStruct((NC * N_PAD,), jnp.float32))
    scratch = (
        pltpu.VMEM((GPT, GROUP), jnp.int32),
        pltpu.VMEM((GPT, GROUP), jnp.int32),
        pltpu.VMEM((GPT, GROUP), jnp.float32),
        pltpu.VMEM((GPT, GROUP), jnp.float32),
        pltpu.VMEM((GROUP, D_HID), jnp.float32),
        pltpu.VMEM((GROUP, D_HID), jnp.float32),
        pltpu.VMEM((GROUP, D_HID), jnp.float32),
        pltpu.VMEM((GROUP, D_HID), jnp.float32),
        pltpu.VMEM_SHARED((N_PAD, D_HID), jnp.float32),
        pltpu.VMEM_SHARED((N_PAD,), jnp.float32),
    ) + (pltpu.SemaphoreType.DMA,) * 9

    def body(feat, src, dst, w, val, z16, z1, acc_out, cnt_out,
             src_v, dst_v, w_v, val_v, rows0, rows1, rows2, rows3,
             acc_sh, cnt_sh, gsem0, gsem1, gsem2, gsem3,
             ssem0, ssem1, ssem2, ssem3, csem):
        _sc_pass(feat, src, dst, w, val, z16, z1, acc_out, cnt_out,
                 src_v, dst_v, w_v, val_v, rows0, rows1, rows2, rows3,
                 acc_sh, cnt_sh, gsem0, gsem1, gsem2, gsem3,
                 ssem0, ssem1, ssem2, ssem3, csem, do_cnt=do_cnt)

    return pl.kernel(body, out_type=out_type, mesh=mesh,
                     scratch_types=scratch,
                     compiler_params=pltpu.CompilerParams(
                         needs_layout_passes=False,
                         use_tc_tiling_on_sc=False))


_sc_pass1 = _make_sc_pass(True)
_sc_pass2 = _make_sc_pass(False)


def _mm_body(x_ref, wl_ref, wr_ref, xl_ref, xr_ref):
    x = x_ref[...]
    xl_ref[...] = jnp.dot(x, wl_ref[...], preferred_element_type=jnp.float32)
    xr_ref[...] = jnp.dot(x, wr_ref[...], preferred_element_type=jnp.float32)


def _h_body(p0_ref, p1_ref, c0_ref, c1_ref, xr_ref, b_ref, h_ref):
    cnt = jnp.maximum(c0_ref[...] + c1_ref[...], 1.0)
    mean = (p0_ref[...] + p1_ref[...]) / cnt
    h_ref[...] = jnp.maximum(mean + xr_ref[...] + b_ref[...], 0.0)


def _out_body(q0_ref, q1_ref, c0_ref, c1_ref, h_ref, wl_ref, wr_ref, b_ref,
              o_ref):
    cnt = jnp.maximum(c0_ref[...] + c1_ref[...], 1.0)
    mean = (q0_ref[...] + q1_ref[...]) / cnt
    o_ref[...] = (jnp.dot(mean, wl_ref[...], preferred_element_type=jnp.float32)
                  + jnp.dot(h_ref[...], wr_ref[...],
                            preferred_element_type=jnp.float32)
                  + b_ref[...])


def _row_spec(width):
    return pl.BlockSpec((M_BLK, width), lambda i: (i, 0))


def _full_spec(shape):
    return pl.BlockSpec(shape, lambda i: (0,) * len(shape))


def kernel(x, edge_index, edge_weight, W_l1, b_l1, W_r1, b_r1,
           W_l2, b_l2, W_r2, b_r2):
    f32 = jnp.float32
    pad = NE_PAD - N_EDGES
    src = jnp.concatenate(
        [edge_index[0].astype(jnp.int32), jnp.zeros((pad,), jnp.int32)]
    ).reshape(NW * GPT, GROUP)
    dst = jnp.concatenate(
        [edge_index[1].astype(jnp.int32), jnp.zeros((pad,), jnp.int32)]
    ).reshape(NW * GPT, GROUP)
    w = jnp.concatenate([edge_weight, jnp.zeros((pad,), f32)]
                        ).reshape(NW * GPT, GROUP)
    val = jnp.concatenate([jnp.ones((N_EDGES,), f32), jnp.zeros((pad,), f32)]
                          ).reshape(NW * GPT, GROUP)
    z16 = jnp.zeros((N_PAD, D_HID), f32)
    z1 = jnp.zeros((N_PAD,), f32)

    grid = N_NODES // M_BLK

    xl, xr = pl.pallas_call(
        _mm_body,
        grid=(grid,),
        in_specs=[_row_spec(D_IN), _full_spec((D_IN, D_HID)),
                  _full_spec((D_IN, D_HID))],
        out_specs=[_row_spec(D_HID), _row_spec(D_HID)],
        out_shape=[jax.ShapeDtypeStruct((N_NODES, D_HID), f32)] * 2,
    )(x, W_l1, W_r1)

    acc1, cnt = _sc_pass1(xl, src, dst, w, val, z16, z1)
    cnt = cnt.reshape(NC, N_PAD)
    c0 = cnt[0, :N_NODES].reshape(N_NODES, 1)
    c1 = cnt[1, :N_NODES].reshape(N_NODES, 1)
    p0 = acc1[0, :N_NODES]
    p1 = acc1[1, :N_NODES]

    b1 = (b_l1 + b_r1).reshape(1, D_HID)
    h = pl.pallas_call(
        _h_body,
        grid=(grid,),
        in_specs=[_row_spec(D_HID), _row_spec(D_HID), _row_spec(1),
                  _row_spec(1), _row_spec(D_HID), _full_spec((1, D_HID))],
        out_specs=_row_spec(D_HID),
        out_shape=jax.ShapeDtypeStruct((N_NODES, D_HID), f32),
    )(p0, p1, c0, c1, xr, b1)

    acc2, _ = _sc_pass2(h, src, dst, w, val, z16, z1)
    q0 = acc2[0, :N_NODES]
    q1 = acc2[1, :N_NODES]

    b2 = (b_l2 + b_r2).reshape(1, D_OUT)
    out = pl.pallas_call(
        _out_body,
        grid=(grid,),
        in_specs=[_row_spec(D_HID), _row_spec(D_HID), _row_spec(1),
                  _row_spec(1), _row_spec(D_HID), _full_spec((D_HID, D_OUT)),
                  _full_spec((D_HID, D_OUT)), _full_spec((1, D_OUT))],
        out_specs=_row_spec(D_OUT),
        out_shape=jax.ShapeDtypeStruct((N_NODES, D_OUT), f32),
    )(q0, q1, c0, c1, h, W_l2, W_r2, b2)

    return out


# speedup vs baseline: 17.8244x; 1.3027x over previous
import functools

import jax
import jax.numpy as jnp
from jax import lax
from jax.experimental import pallas as pl
from jax.experimental.pallas import tpu as pltpu
from jax.experimental.pallas import tpu_sc as plsc

N_NODES = 10000
N_EDGES = 320000
D_IN = 128
D_HID = 16
D_OUT = 128

NC = 2
NS = 16
NW = NC * NS
GROUP = 128
GPT = 80
NE_PAD = NW * GPT * GROUP
N_PAD = 10240
ROWS_PER_SUB = N_PAD // NS
M_BLK = 1000


NBUF = 4


def _sc_pass(feat_hbm, src_hbm, dst_hbm, w_hbm, val_hbm, z16_hbm, z1_hbm,
             acc_out, cnt_out, src_v, dst_v, w_v, val_v,
             rows0, rows1, rows2, rows3, acc_sh, cnt_sh,
             gsem0, gsem1, gsem2, gsem3, ssem0, ssem1, ssem2, ssem3, csem,
             *, do_cnt):
    c = lax.axis_index("c")
    s = lax.axis_index("s")
    wid = c * NS + s
    rows = (rows0, rows1, rows2, rows3)
    gsem = (gsem0, gsem1, gsem2, gsem3)
    ssem = (ssem0, ssem1, ssem2, ssem3)

    pltpu.sync_copy(z16_hbm.at[pl.ds(s * ROWS_PER_SUB, ROWS_PER_SUB)],
                    acc_sh.at[pl.ds(s * ROWS_PER_SUB, ROWS_PER_SUB)])
    if do_cnt:
        @pl.when(s == 0)
        def _():
            pltpu.sync_copy(z1_hbm, cnt_sh)

    base = wid * GPT
    pltpu.sync_copy(src_hbm.at[pl.ds(base, GPT)], src_v)
    pltpu.sync_copy(dst_hbm.at[pl.ds(base, GPT)], dst_v)
    pltpu.sync_copy(w_hbm.at[pl.ds(base, GPT)], w_v)
    if do_cnt:
        pltpu.sync_copy(val_hbm.at[pl.ds(base, GPT)], val_v)
    plsc.subcore_barrier()

    iota = lax.iota(jnp.int32, 16)

    def gather_start(j, b):
        pltpu.async_copy(feat_hbm.at[src_v.at[j]], rows[b], gsem[b])

    def gather_wait(j, b):
        pltpu.make_async_copy(feat_hbm.at[src_v.at[j]], rows[b],
                              gsem[b]).wait()

    def scatter_start(j, b):
        pltpu.async_copy(rows[b], acc_sh.at[dst_v.at[j]], ssem[b], add=True)

    def scatter_wait(j, b):
        pltpu.make_async_copy(rows[b], acc_sh.at[dst_v.at[j]],
                              ssem[b]).wait()

    def scale(j, b):
        rv = rows[b]
        jj = jnp.full((16,), j, dtype=jnp.int32)

        @plsc.parallel_loop(0, GROUP, 1, unroll=8)
        def _(i):
            ii = jnp.full((16,), i, dtype=jnp.int32)
            wsplat = plsc.load_gather(w_v, [jj, ii])
            row = plsc.load_gather(rv, [ii, iota])
            plsc.store_scatter(rv, [ii, iota], row * wsplat)

    gather_start(0, 0)
    gather_start(1, 1)

    def handle_group(j4, u):
        j = NBUF * j4 + u
        b = (u + 2) % NBUF
        gather_wait(j, u)
        if u < 2:
            @pl.when(j4 > 0)
            def _():
                scatter_wait(j - 2, b)
            gather_start(j + 2, b)
        else:
            @pl.when(j + 2 < GPT)
            def _():
                scatter_wait(j - 2, b)
                gather_start(j + 2, b)
        scale(j, u)
        scatter_start(j, u)
        if do_cnt:
            pltpu.async_copy(val_v.at[j], cnt_sh.at[dst_v.at[j]], csem,
                             add=True)

    def outer_body(j4, _):
        for u in range(NBUF):
            handle_group(j4, u)
        return 0

    lax.fori_loop(0, GPT // NBUF, outer_body, 0)
    for jt in range(GPT - NBUF, GPT):
        scatter_wait(jt, jt % NBUF)
    if do_cnt:
        def cnt_drain(j, _):
            pltpu.make_async_copy(val_v.at[j], cnt_sh.at[dst_v.at[j]],
                                  csem).wait()
            return 0
        lax.fori_loop(0, GPT, cnt_drain, 0)
    plsc.subcore_barrier()

    pltpu.sync_copy(acc_sh.at[pl.ds(s * ROWS_PER_SUB, ROWS_PER_SUB)],
                    acc_out.at[c, pl.ds(s * ROWS_PER_SUB, ROWS_PER_SUB)])
    if do_cnt:
        @pl.when(s == 0)
        def _():
            pltpu.sync_copy(cnt_sh, cnt_out.at[pl.ds(c * N_PAD, N_PAD)])


def _sc_pass2_body(acc1_hbm, cnt_hbm, xr_hbm, b1_hbm, src_hbm, dst_hbm,
                   w_hbm, z16_hbm, acc_out, h_out,
                   src_v, dst_v, w_v, a0_v, a1_v, c0_v, c1_v, xr_v, b_v, h_v,
                   rows0, rows1, rows2, rows3, acc_sh, h_sh,
                   gsem0, gsem1, gsem2, gsem3, ssem0, ssem1, ssem2, ssem3):
    c = lax.axis_index("c")
    s = lax.axis_index("s")
    wid = c * NS + s
    rows = (rows0, rows1, rows2, rows3)
    gsem = (gsem0, gsem1, gsem2, gsem3)
    ssem = (ssem0, ssem1, ssem2, ssem3)
    iota = lax.iota(jnp.int32, 16)
    base_r = s * ROWS_PER_SUB

    pltpu.sync_copy(z16_hbm.at[pl.ds(base_r, ROWS_PER_SUB)],
                    acc_sh.at[pl.ds(base_r, ROWS_PER_SUB)])
    pltpu.sync_copy(acc1_hbm.at[0, pl.ds(base_r, ROWS_PER_SUB)], a0_v)
    pltpu.sync_copy(acc1_hbm.at[1, pl.ds(base_r, ROWS_PER_SUB)], a1_v)
    pltpu.sync_copy(cnt_hbm.at[pl.ds(base_r, ROWS_PER_SUB)], c0_v)
    pltpu.sync_copy(cnt_hbm.at[pl.ds(N_PAD + base_r, ROWS_PER_SUB)], c1_v)
    pltpu.sync_copy(xr_hbm.at[pl.ds(base_r, ROWS_PER_SUB)], xr_v)
    pltpu.sync_copy(b1_hbm, b_v)
    bias = b_v[...]

    @plsc.parallel_loop(0, ROWS_PER_SUB, 1, unroll=4)
    def _(r):
        rr = jnp.full((16,), r, dtype=jnp.int32)
        a = plsc.load_gather(a0_v, [rr, iota]) + plsc.load_gather(a1_v, [rr, iota])
        cs = plsc.load_gather(c0_v, [rr]) + plsc.load_gather(c1_v, [rr])
        mean = a / jnp.maximum(cs, 1.0)
        hrow = jnp.maximum(mean + plsc.load_gather(xr_v, [rr, iota]) + bias,
                           0.0)
        plsc.store_scatter(h_v, [rr, iota], hrow)

    pltpu.sync_copy(h_v, h_sh.at[pl.ds(base_r, ROWS_PER_SUB)])

    @pl.when(c == 0)
    def _():
        pltpu.sync_copy(h_v, h_out.at[pl.ds(base_r, ROWS_PER_SUB)])

    base = wid * GPT
    pltpu.sync_copy(src_hbm.at[pl.ds(base, GPT)], src_v)
    pltpu.sync_copy(dst_hbm.at[pl.ds(base, GPT)], dst_v)
    pltpu.sync_copy(w_hbm.at[pl.ds(base, GPT)], w_v)
    plsc.subcore_barrier()

    def gather_start(j, b):
        pltpu.async_copy(h_sh.at[src_v.at[j]], rows[b], gsem[b])

    def gather_wait(j, b):
        pltpu.make_async_copy(h_sh.at[src_v.at[j]], rows[b], gsem[b]).wait()

    def scatter_start(j, b):
        pltpu.async_copy(rows[b], acc_sh.at[dst_v.at[j]], ssem[b], add=True)

    def scatter_wait(j, b):
        pltpu.make_async_copy(rows[b], acc_sh.at[dst_v.at[j]],
                              ssem[b]).wait()

    def scale(j, b):
        rv = rows[b]
        jj = jnp.full((16,), j, dtype=jnp.int32)

        @plsc.parallel_loop(0, GROUP, 1, unroll=8)
        def _(i):
            ii = jnp.full((16,), i, dtype=jnp.int32)
            wsplat = plsc.load_gather(w_v, [jj, ii])
            row = plsc.load_gather(rv, [ii, iota])
            plsc.store_scatter(rv, [ii, iota], row * wsplat)

    gather_start(0, 0)
    gather_start(1, 1)

    def handle_group(j4, u):
        j = NBUF * j4 + u
        b = (u + 2) % NBUF
        gather_wait(j, u)
        if u < 2:
            @pl.when(j4 > 0)
            def _():
                scatter_wait(j - 2, b)
            gather_start(j + 2, b)
        else:
            @pl.when(j + 2 < GPT)
            def _():
                scatter_wait(j - 2, b)
                gather_start(j + 2, b)
        scale(j, u)
        scatter_start(j, u)

    def outer_body(j4, _):
        for u in range(NBUF):
            handle_group(j4, u)
        return 0

    lax.fori_loop(0, GPT // NBUF, outer_body, 0)
    for jt in range(GPT - NBUF, GPT):
        scatter_wait(jt, jt % NBUF)
    plsc.subcore_barrier()

    pltpu.sync_copy(acc_sh.at[pl.ds(base_r, ROWS_PER_SUB)],
                    acc_out.at[c, pl.ds(base_r, ROWS_PER_SUB)])


def _make_sc_pass2():
    mesh = plsc.VectorSubcoreMesh(core_axis_name="c", subcore_axis_name="s",
                                  num_cores=NC, num_subcores=NS)
    out_type = (jax.ShapeDtypeStruct((NC, N_PAD, D_HID), jnp.float32),
                jax.ShapeDtypeStruct((N_PAD, D_HID), jnp.float32))
    scratch = (
        pltpu.VMEM((GPT, GROUP), jnp.int32),
        pltpu.VMEM((GPT, GROUP), jnp.int32),
        pltpu.VMEM((GPT, GROUP), jnp.float32),
        pltpu.VMEM((ROWS_PER_SUB, D_HID), jnp.float32),
        pltpu.VMEM((ROWS_PER_SUB, D_HID), jnp.float32),
        pltpu.VMEM((ROWS_PER_SUB,), jnp.float32),
        pltpu.VMEM((ROWS_PER_SUB,), jnp.float32),
        pltpu.VMEM((ROWS_PER_SUB, D_HID), jnp.float32),
        pltpu.VMEM((D_HID,), jnp.float32),
        pltpu.VMEM((ROWS_PER_SUB, D_HID), jnp.float32),
        pltpu.VMEM((GROUP, D_HID), jnp.float32),
        pltpu.VMEM((GROUP, D_HID), jnp.float32),
        pltpu.VMEM((GROUP, D_HID), jnp.float32),
        pltpu.VMEM((GROUP, D_HID), jnp.float32),
        pltpu.VMEM_SHARED((N_PAD, D_HID), jnp.float32),
        pltpu.VMEM_SHARED((N_PAD, D_HID), jnp.float32),
    ) + (pltpu.SemaphoreType.DMA,) * 8

    return pl.kernel(_sc_pass2_body, out_type=out_type, mesh=mesh,
                     scratch_types=scratch,
                     compiler_params=pltpu.CompilerParams(
                         needs_layout_passes=False,
                         use_tc_tiling_on_sc=False))


def _make_sc_pass(do_cnt):
    mesh = plsc.VectorSubcoreMesh(core_axis_name="c", subcore_axis_name="s",
                                  num_cores=NC, num_subcores=NS)
    out_type = (jax.ShapeDtypeStruct((NC, N_PAD, D_HID), jnp.float32),
                jax.ShapeDtypeStruct((NC * N_PAD,), jnp.float32))
    scratch = (
        pltpu.VMEM((GPT, GROUP), jnp.int32),
        pltpu.VMEM((GPT, GROUP), jnp.int32),
        pltpu.VMEM((GPT, GROUP), jnp.float32),
        pltpu.VMEM((GPT, GROUP), jnp.float32),
        pltpu.VMEM((GROUP, D_HID), jnp.float32),
        pltpu.VMEM((GROUP, D_HID), jnp.float32),
        pltpu.VMEM((GROUP, D_HID), jnp.float32),
        pltpu.VMEM((GROUP, D_HID), jnp.float32),
        pltpu.VMEM_SHARED((N_PAD, D_HID), jnp.float32),
        pltpu.VMEM_SHARED((N_PAD,), jnp.float32),
    ) + (pltpu.SemaphoreType.DMA,) * 9

    def body(feat, src, dst, w, val, z16, z1, acc_out, cnt_out,
             src_v, dst_v, w_v, val_v, rows0, rows1, rows2, rows3,
             acc_sh, cnt_sh, gsem0, gsem1, gsem2, gsem3,
             ssem0, ssem1, ssem2, ssem3, csem):
        _sc_pass(feat, src, dst, w, val, z16, z1, acc_out, cnt_out,
                 src_v, dst_v, w_v, val_v, rows0, rows1, rows2, rows3,
                 acc_sh, cnt_sh, gsem0, gsem1, gsem2, gsem3,
                 ssem0, ssem1, ssem2, ssem3, csem, do_cnt=do_cnt)

    return pl.kernel(body, out_type=out_type, mesh=mesh,
                     scratch_types=scratch,
                     compiler_params=pltpu.CompilerParams(
                         needs_layout_passes=False,
                         use_tc_tiling_on_sc=False))


_sc_pass1 = _make_sc_pass(True)
_sc_pass2 = _make_sc_pass2()


def _mm_body(x_ref, wl_ref, wr_ref, xl_ref, xr_ref):
    x = x_ref[...]
    xl_ref[...] = jnp.dot(x, wl_ref[...], preferred_element_type=jnp.float32)
    xr_ref[...] = jnp.dot(x, wr_ref[...], preferred_element_type=jnp.float32)


def _out_body(q0_ref, q1_ref, c0_ref, c1_ref, h_ref, wl_ref, wr_ref, b_ref,
              o_ref):
    cnt = jnp.maximum(c0_ref[...] + c1_ref[...], 1.0)
    mean = (q0_ref[...] + q1_ref[...]) / cnt
    o_ref[...] = (jnp.dot(mean, wl_ref[...], preferred_element_type=jnp.float32)
                  + jnp.dot(h_ref[...], wr_ref[...],
                            preferred_element_type=jnp.float32)
                  + b_ref[...])


def _row_spec(width):
    return pl.BlockSpec((M_BLK, width), lambda i: (i, 0))


def _full_spec(shape):
    return pl.BlockSpec(shape, lambda i: (0,) * len(shape))


def kernel(x, edge_index, edge_weight, W_l1, b_l1, W_r1, b_r1,
           W_l2, b_l2, W_r2, b_r2):
    f32 = jnp.float32
    pad = NE_PAD - N_EDGES
    src = jnp.concatenate(
        [edge_index[0].astype(jnp.int32), jnp.zeros((pad,), jnp.int32)]
    ).reshape(NW * GPT, GROUP)
    dst = jnp.concatenate(
        [edge_index[1].astype(jnp.int32), jnp.zeros((pad,), jnp.int32)]
    ).reshape(NW * GPT, GROUP)
    w = jnp.concatenate([edge_weight, jnp.zeros((pad,), f32)]
                        ).reshape(NW * GPT, GROUP)
    val = jnp.concatenate([jnp.ones((N_EDGES,), f32), jnp.zeros((pad,), f32)]
                          ).reshape(NW * GPT, GROUP)
    z16 = jnp.zeros((N_PAD, D_HID), f32)
    z1 = jnp.zeros((N_PAD,), f32)
    xp = jnp.pad(x, ((0, N_PAD - N_NODES), (0, 0)))

    grid = N_NODES // M_BLK
    grid_a = N_PAD // 1024

    xl, xr = pl.pallas_call(
        _mm_body,
        grid=(grid_a,),
        in_specs=[pl.BlockSpec((1024, D_IN), lambda i: (i, 0)),
                  _full_spec((D_IN, D_HID)), _full_spec((D_IN, D_HID))],
        out_specs=[pl.BlockSpec((1024, D_HID), lambda i: (i, 0))] * 2,
        out_shape=[jax.ShapeDtypeStruct((N_PAD, D_HID), f32)] * 2,
    )(xp, W_l1, W_r1)

    acc1, cnt = _sc_pass1(xl, src, dst, w, val, z16, z1)
    b1 = b_l1 + b_r1

    acc2, h = _sc_pass2(acc1, cnt, xr, b1, src, dst, w, z16)
    h = h[:N_NODES]
    cnt = cnt.reshape(NC, N_PAD)
    c0 = cnt[0, :N_NODES].reshape(N_NODES, 1)
    c1 = cnt[1, :N_NODES].reshape(N_NODES, 1)
    q0 = acc2[0, :N_NODES]
    q1 = acc2[1, :N_NODES]

    b2 = (b_l2 + b_r2).reshape(1, D_OUT)
    out = pl.pallas_call(
        _out_body,
        grid=(grid,),
        in_specs=[_row_spec(D_HID), _row_spec(D_HID), _row_spec(1),
                  _row_spec(1), _row_spec(D_HID), _full_spec((D_HID, D_OUT)),
                  _full_spec((D_HID, D_OUT)), _full_spec((1, D_OUT))],
        out_specs=_row_spec(D_OUT),
        out_shape=jax.ShapeDtypeStruct((N_NODES, D_OUT), f32),
    )(q0, q1, c0, c1, h, W_l2, W_r2, b2)

    return out


# speedup vs baseline: 22.1478x; 1.2426x over previous
import functools

import jax
import jax.numpy as jnp
from jax import lax
from jax.experimental import pallas as pl
from jax.experimental.pallas import tpu as pltpu
from jax.experimental.pallas import tpu_sc as plsc

N_NODES = 10000
N_EDGES = 320000
D_IN = 128
D_HID = 16
D_OUT = 128

NC = 2
NS = 16
NW = NC * NS
GROUP = 128
GPT = 80
NE_PAD = NW * GPT * GROUP
N_PAD = 10240
ROWS_PER_SUB = N_PAD // NS
M_BLK = 1000


NBUF = 4


def _sc_pass(feat_hbm, src_hbm, dst_hbm, w_hbm, z16_hbm, z1_hbm,
             acc_out, cnt_out, src_v, dst_v, w_v, ones_v,
             rows0, rows1, rows2, rows3, acc_sh, cnt_sh, feat_sh,
             gsem0, gsem1, gsem2, gsem3, ssem0, ssem1, ssem2, ssem3, csem,
             *, do_cnt):
    c = lax.axis_index("c")
    s = lax.axis_index("s")
    wid = c * NS + s
    rows = (rows0, rows1, rows2, rows3)
    gsem = (gsem0, gsem1, gsem2, gsem3)
    ssem = (ssem0, ssem1, ssem2, ssem3)
    base_r = s * ROWS_PER_SUB

    pltpu.sync_copy(z16_hbm.at[pl.ds(base_r, ROWS_PER_SUB)],
                    acc_sh.at[pl.ds(base_r, ROWS_PER_SUB)])
    pltpu.sync_copy(feat_hbm.at[pl.ds(base_r, ROWS_PER_SUB)],
                    feat_sh.at[pl.ds(base_r, ROWS_PER_SUB)])
    if do_cnt:
        @pl.when(s == 0)
        def _():
            pltpu.sync_copy(z1_hbm, cnt_sh)
        for k in range(GROUP // 16):
            ones_v[0, pl.ds(k * 16, 16)] = jnp.ones((16,), jnp.float32)

    base = wid * GPT
    pltpu.sync_copy(src_hbm.at[pl.ds(base, GPT)], src_v)
    pltpu.sync_copy(dst_hbm.at[pl.ds(base, GPT)], dst_v)
    pltpu.sync_copy(w_hbm.at[pl.ds(base, GPT)], w_v)
    plsc.subcore_barrier()

    iota = lax.iota(jnp.int32, 16)

    def gather_start(j, b):
        pltpu.async_copy(feat_sh.at[src_v.at[j]], rows[b], gsem[b])

    def gather_wait(j, b):
        pltpu.make_async_copy(feat_sh.at[src_v.at[j]], rows[b],
                              gsem[b]).wait()

    def scatter_start(j, b):
        pltpu.async_copy(rows[b], acc_sh.at[dst_v.at[j]], ssem[b], add=True)

    def scatter_wait(j, b):
        pltpu.make_async_copy(rows[b], acc_sh.at[dst_v.at[j]],
                              ssem[b]).wait()

    def scale(j, b):
        rv = rows[b]
        jj = jnp.full((16,), j, dtype=jnp.int32)

        @plsc.parallel_loop(0, GROUP, 1, unroll=8)
        def _(i):
            ii = jnp.full((16,), i, dtype=jnp.int32)
            wsplat = plsc.load_gather(w_v, [jj, ii])
            row = plsc.load_gather(rv, [ii, iota])
            plsc.store_scatter(rv, [ii, iota], row * wsplat)

    gather_start(0, 0)
    gather_start(1, 1)

    def handle_group(j4, u):
        j = NBUF * j4 + u
        b = (u + 2) % NBUF
        gather_wait(j, u)
        if u < 2:
            @pl.when(j4 > 0)
            def _():
                scatter_wait(j - 2, b)
            gather_start(j + 2, b)
        else:
            @pl.when(j + 2 < GPT)
            def _():
                scatter_wait(j - 2, b)
                gather_start(j + 2, b)
        scale(j, u)
        scatter_start(j, u)
        if do_cnt:
            pltpu.async_copy(ones_v.at[0], cnt_sh.at[dst_v.at[j]], csem,
                             add=True)

    def outer_body(j4, _):
        for u in range(NBUF):
            handle_group(j4, u)
        return 0

    lax.fori_loop(0, GPT // NBUF, outer_body, 0)
    for jt in range(GPT - NBUF, GPT):
        scatter_wait(jt, jt % NBUF)
    if do_cnt:
        def cnt_drain(j, _):
            pltpu.make_async_copy(ones_v.at[0], cnt_sh.at[dst_v.at[j]],
                                  csem).wait()
            return 0
        lax.fori_loop(0, GPT, cnt_drain, 0)
    plsc.subcore_barrier()

    pltpu.sync_copy(acc_sh.at[pl.ds(base_r, ROWS_PER_SUB)],
                    acc_out.at[c, pl.ds(base_r, ROWS_PER_SUB)])
    if do_cnt:
        @pl.when(s == 0)
        def _():
            pltpu.sync_copy(cnt_sh, cnt_out.at[pl.ds(c * N_PAD, N_PAD)])


def _sc_pass2_body(acc1_hbm, cnt_hbm, xr_hbm, b1_hbm, src_hbm, dst_hbm,
                   w_hbm, z16_hbm, acc_out, h_out,
                   src_v, dst_v, w_v, a0_v, a1_v, c0_v, c1_v, xr_v, b_v, h_v,
                   rows0, rows1, rows2, rows3, acc_sh, h_sh,
                   gsem0, gsem1, gsem2, gsem3, ssem0, ssem1, ssem2, ssem3):
    c = lax.axis_index("c")
    s = lax.axis_index("s")
    wid = c * NS + s
    rows = (rows0, rows1, rows2, rows3)
    gsem = (gsem0, gsem1, gsem2, gsem3)
    ssem = (ssem0, ssem1, ssem2, ssem3)
    iota = lax.iota(jnp.int32, 16)
    base_r = s * ROWS_PER_SUB

    pltpu.sync_copy(z16_hbm.at[pl.ds(base_r, ROWS_PER_SUB)],
                    acc_sh.at[pl.ds(base_r, ROWS_PER_SUB)])
    pltpu.sync_copy(acc1_hbm.at[0, pl.ds(base_r, ROWS_PER_SUB)], a0_v)
    pltpu.sync_copy(acc1_hbm.at[1, pl.ds(base_r, ROWS_PER_SUB)], a1_v)
    pltpu.sync_copy(cnt_hbm.at[pl.ds(base_r, ROWS_PER_SUB)], c0_v)
    pltpu.sync_copy(cnt_hbm.at[pl.ds(N_PAD + base_r, ROWS_PER_SUB)], c1_v)
    pltpu.sync_copy(xr_hbm.at[pl.ds(base_r, ROWS_PER_SUB)], xr_v)
    pltpu.sync_copy(b1_hbm, b_v)
    bias = b_v[...]

    @plsc.parallel_loop(0, ROWS_PER_SUB, 1, unroll=4)
    def _(r):
        rr = jnp.full((16,), r, dtype=jnp.int32)
        a = plsc.load_gather(a0_v, [rr, iota]) + plsc.load_gather(a1_v, [rr, iota])
        cs = plsc.load_gather(c0_v, [rr]) + plsc.load_gather(c1_v, [rr])
        mean = a / jnp.maximum(cs, 1.0)
        hrow = jnp.maximum(mean + plsc.load_gather(xr_v, [rr, iota]) + bias,
                           0.0)
        plsc.store_scatter(h_v, [rr, iota], hrow)

    pltpu.sync_copy(h_v, h_sh.at[pl.ds(base_r, ROWS_PER_SUB)])

    @pl.when(c == 0)
    def _():
        pltpu.sync_copy(h_v, h_out.at[pl.ds(base_r, ROWS_PER_SUB)])

    base = wid * GPT
    pltpu.sync_copy(src_hbm.at[pl.ds(base, GPT)], src_v)
    pltpu.sync_copy(dst_hbm.at[pl.ds(base, GPT)], dst_v)
    pltpu.sync_copy(w_hbm.at[pl.ds(base, GPT)], w_v)
    plsc.subcore_barrier()

    def gather_start(j, b):
        pltpu.async_copy(h_sh.at[src_v.at[j]], rows[b], gsem[b])

    def gather_wait(j, b):
        pltpu.make_async_copy(h_sh.at[src_v.at[j]], rows[b], gsem[b]).wait()

    def scatter_start(j, b):
        pltpu.async_copy(rows[b], acc_sh.at[dst_v.at[j]], ssem[b], add=True)

    def scatter_wait(j, b):
        pltpu.make_async_copy(rows[b], acc_sh.at[dst_v.at[j]],
                              ssem[b]).wait()

    def scale(j, b):
        rv = rows[b]
        jj = jnp.full((16,), j, dtype=jnp.int32)

        @plsc.parallel_loop(0, GROUP, 1, unroll=8)
        def _(i):
            ii = jnp.full((16,), i, dtype=jnp.int32)
            wsplat = plsc.load_gather(w_v, [jj, ii])
            row = plsc.load_gather(rv, [ii, iota])
            plsc.store_scatter(rv, [ii, iota], row * wsplat)

    gather_start(0, 0)
    gather_start(1, 1)

    def handle_group(j4, u):
        j = NBUF * j4 + u
        b = (u + 2) % NBUF
        gather_wait(j, u)
        if u < 2:
            @pl.when(j4 > 0)
            def _():
                scatter_wait(j - 2, b)
            gather_start(j + 2, b)
        else:
            @pl.when(j + 2 < GPT)
            def _():
                scatter_wait(j - 2, b)
                gather_start(j + 2, b)
        scale(j, u)
        scatter_start(j, u)

    def outer_body(j4, _):
        for u in range(NBUF):
            handle_group(j4, u)
        return 0

    lax.fori_loop(0, GPT // NBUF, outer_body, 0)
    for jt in range(GPT - NBUF, GPT):
        scatter_wait(jt, jt % NBUF)
    plsc.subcore_barrier()

    pltpu.sync_copy(acc_sh.at[pl.ds(base_r, ROWS_PER_SUB)],
                    acc_out.at[c, pl.ds(base_r, ROWS_PER_SUB)])


def _make_sc_pass2():
    mesh = plsc.VectorSubcoreMesh(core_axis_name="c", subcore_axis_name="s",
                                  num_cores=NC, num_subcores=NS)
    out_type = (jax.ShapeDtypeStruct((NC, N_PAD, D_HID), jnp.float32),
                jax.ShapeDtypeStruct((N_PAD, D_HID), jnp.float32))
    scratch = (
        pltpu.VMEM((GPT, GROUP), jnp.int32),
        pltpu.VMEM((GPT, GROUP), jnp.int32),
        pltpu.VMEM((GPT, GROUP), jnp.float32),
        pltpu.VMEM((ROWS_PER_SUB, D_HID), jnp.float32),
        pltpu.VMEM((ROWS_PER_SUB, D_HID), jnp.float32),
        pltpu.VMEM((ROWS_PER_SUB,), jnp.float32),
        pltpu.VMEM((ROWS_PER_SUB,), jnp.float32),
        pltpu.VMEM((ROWS_PER_SUB, D_HID), jnp.float32),
        pltpu.VMEM((D_HID,), jnp.float32),
        pltpu.VMEM((ROWS_PER_SUB, D_HID), jnp.float32),
        pltpu.VMEM((GROUP, D_HID), jnp.float32),
        pltpu.VMEM((GROUP, D_HID), jnp.float32),
        pltpu.VMEM((GROUP, D_HID), jnp.float32),
        pltpu.VMEM((GROUP, D_HID), jnp.float32),
        pltpu.VMEM_SHARED((N_PAD, D_HID), jnp.float32),
        pltpu.VMEM_SHARED((N_PAD, D_HID), jnp.float32),
    ) + (pltpu.SemaphoreType.DMA,) * 8

    return pl.kernel(_sc_pass2_body, out_type=out_type, mesh=mesh,
                     scratch_types=scratch,
                     compiler_params=pltpu.CompilerParams(
                         needs_layout_passes=False,
                         use_tc_tiling_on_sc=False))


def _make_sc_pass(do_cnt):
    mesh = plsc.VectorSubcoreMesh(core_axis_name="c", subcore_axis_name="s",
                                  num_cores=NC, num_subcores=NS)
    out_type = (jax.ShapeDtypeStruct((NC, N_PAD, D_HID), jnp.float32),
                jax.ShapeDtypeStruct((NC * N_PAD,), jnp.float32))
    scratch = (
        pltpu.VMEM((GPT, GROUP), jnp.int32),
        pltpu.VMEM((GPT, GROUP), jnp.int32),
        pltpu.VMEM((GPT, GROUP), jnp.float32),
        pltpu.VMEM((1, GROUP), jnp.float32),
        pltpu.VMEM((GROUP, D_HID), jnp.float32),
        pltpu.VMEM((GROUP, D_HID), jnp.float32),
        pltpu.VMEM((GROUP, D_HID), jnp.float32),
        pltpu.VMEM((GROUP, D_HID), jnp.float32),
        pltpu.VMEM_SHARED((N_PAD, D_HID), jnp.float32),
        pltpu.VMEM_SHARED((N_PAD,), jnp.float32),
        pltpu.VMEM_SHARED((N_PAD, D_HID), jnp.float32),
    ) + (pltpu.SemaphoreType.DMA,) * 9

    def body(feat, src, dst, w, z16, z1, acc_out, cnt_out,
             src_v, dst_v, w_v, ones_v, rows0, rows1, rows2, rows3,
             acc_sh, cnt_sh, feat_sh, gsem0, gsem1, gsem2, gsem3,
             ssem0, ssem1, ssem2, ssem3, csem):
        _sc_pass(feat, src, dst, w, z16, z1, acc_out, cnt_out,
                 src_v, dst_v, w_v, ones_v, rows0, rows1, rows2, rows3,
                 acc_sh, cnt_sh, feat_sh, gsem0, gsem1, gsem2, gsem3,
                 ssem0, ssem1, ssem2, ssem3, csem, do_cnt=do_cnt)

    return pl.kernel(body, out_type=out_type, mesh=mesh,
                     scratch_types=scratch,
                     compiler_params=pltpu.CompilerParams(
                         needs_layout_passes=False,
                         use_tc_tiling_on_sc=False))


_sc_pass1 = _make_sc_pass(True)
_sc_pass2 = _make_sc_pass2()


def _mm_body(x_ref, wl_ref, wr_ref, xl_ref, xr_ref):
    x = x_ref[...]
    xl_ref[...] = jnp.dot(x, wl_ref[...], preferred_element_type=jnp.float32)
    xr_ref[...] = jnp.dot(x, wr_ref[...], preferred_element_type=jnp.float32)


def _out_body(q0_ref, q1_ref, c0_ref, c1_ref, h_ref, wl_ref, wr_ref, b_ref,
              o_ref):
    cnt = jnp.maximum(c0_ref[...] + c1_ref[...], 1.0)
    mean = (q0_ref[...] + q1_ref[...]) / cnt
    o_ref[...] = (jnp.dot(mean, wl_ref[...], preferred_element_type=jnp.float32)
                  + jnp.dot(h_ref[...], wr_ref[...],
                            preferred_element_type=jnp.float32)
                  + b_ref[...])


def _row_spec(width):
    return pl.BlockSpec((M_BLK, width), lambda i: (i, 0))


def _full_spec(shape):
    return pl.BlockSpec(shape, lambda i: (0,) * len(shape))


def kernel(x, edge_index, edge_weight, W_l1, b_l1, W_r1, b_r1,
           W_l2, b_l2, W_r2, b_r2):
    f32 = jnp.float32
    pad = NE_PAD - N_EDGES
    src = jnp.concatenate(
        [edge_index[0].astype(jnp.int32), jnp.zeros((pad,), jnp.int32)]
    ).reshape(NW * GPT, GROUP)
    dst = jnp.concatenate(
        [edge_index[1].astype(jnp.int32),
         jnp.full((pad,), N_PAD - 1, jnp.int32)]
    ).reshape(NW * GPT, GROUP)
    w = jnp.concatenate([edge_weight, jnp.zeros((pad,), f32)]
                        ).reshape(NW * GPT, GROUP)
    z16 = jnp.zeros((N_PAD, D_HID), f32)
    z1 = jnp.zeros((N_PAD,), f32)
    xp = jnp.pad(x, ((0, N_PAD - N_NODES), (0, 0)))

    grid = N_NODES // M_BLK
    grid_a = N_PAD // 1024

    xl, xr = pl.pallas_call(
        _mm_body,
        grid=(grid_a,),
        in_specs=[pl.BlockSpec((1024, D_IN), lambda i: (i, 0)),
                  _full_spec((D_IN, D_HID)), _full_spec((D_IN, D_HID))],
        out_specs=[pl.BlockSpec((1024, D_HID), lambda i: (i, 0))] * 2,
        out_shape=[jax.ShapeDtypeStruct((N_PAD, D_HID), f32)] * 2,
    )(xp, W_l1, W_r1)

    acc1, cnt = _sc_pass1(xl, src, dst, w, z16, z1)
    b1 = b_l1 + b_r1

    acc2, h = _sc_pass2(acc1, cnt, xr, b1, src, dst, w, z16)
    h = h[:N_NODES]
    cnt = cnt.reshape(NC, N_PAD)
    c0 = cnt[0, :N_NODES].reshape(N_NODES, 1)
    c1 = cnt[1, :N_NODES].reshape(N_NODES, 1)
    q0 = acc2[0, :N_NODES]
    q1 = acc2[1, :N_NODES]

    b2 = (b_l2 + b_r2).reshape(1, D_OUT)
    out = pl.pallas_call(
        _out_body,
        grid=(grid,),
        in_specs=[_row_spec(D_HID), _row_spec(D_HID), _row_spec(1),
                  _row_spec(1), _row_spec(D_HID), _full_spec((D_HID, D_OUT)),
                  _full_spec((D_HID, D_OUT)), _full_spec((1, D_OUT))],
        out_specs=_row_spec(D_OUT),
        out_shape=jax.ShapeDtypeStruct((N_NODES, D_OUT), f32),
    )(q0, q1, c0, c1, h, W_l2, W_r2, b2)

    return out


# speedup vs baseline: 25.9617x; 1.1722x over previous
import functools

import jax
import jax.numpy as jnp
from jax import lax
from jax.experimental import pallas as pl
from jax.experimental.pallas import tpu as pltpu
from jax.experimental.pallas import tpu_sc as plsc

N_NODES = 10000
N_EDGES = 320000
D_IN = 128
D_HID = 16
D_OUT = 128

NC = 2
NS = 16
NW = NC * NS
GROUP = 128
GPT = 80
NE_PAD = NW * GPT * GROUP
N_PAD = 10240
ROWS_PER_SUB = N_PAD // NS
M_BLK = 1000


NBUF = 4


def _zero_slab_via(buf, target_sh, base_r):
    for k in range(GROUP):
        buf[k, :] = jnp.zeros((D_HID,), jnp.float32)
    for k in range(ROWS_PER_SUB // GROUP):
        pltpu.sync_copy(buf, target_sh.at[pl.ds(base_r + k * GROUP, GROUP)])


def _sc_pass(feat_hbm, e2_hbm, w_hbm, acc_out, cnt_out,
             src_v, dst_v, w_v, ones_v,
             rows0, rows1, rows2, rows3, acc_sh, cnt_sh, feat_sh,
             gsem0, gsem1, gsem2, gsem3, ssem0, ssem1, ssem2, ssem3, csem,
             *, do_cnt):
    c = lax.axis_index("c")
    s = lax.axis_index("s")
    wid = c * NS + s
    rows = (rows0, rows1, rows2, rows3)
    gsem = (gsem0, gsem1, gsem2, gsem3)
    ssem = (ssem0, ssem1, ssem2, ssem3)
    base_r = s * ROWS_PER_SUB

    _zero_slab_via(rows0, acc_sh, base_r)
    pltpu.sync_copy(feat_hbm.at[pl.ds(base_r, ROWS_PER_SUB)],
                    feat_sh.at[pl.ds(base_r, ROWS_PER_SUB)])
    if do_cnt:
        for k in range(GROUP // 16):
            ones_v[0, pl.ds(k * 16, 16)] = jnp.ones((16,), jnp.float32)
            ones_v[1, pl.ds(k * 16, 16)] = jnp.zeros((16,), jnp.float32)
        for k in range(ROWS_PER_SUB // GROUP):
            pltpu.sync_copy(ones_v.at[1],
                            cnt_sh.at[pl.ds(base_r + k * GROUP, GROUP)])

    base = wid * GPT
    pltpu.sync_copy(e2_hbm.at[0, pl.ds(base, GPT)], src_v)
    pltpu.sync_copy(e2_hbm.at[1, pl.ds(base, GPT)], dst_v)
    pltpu.sync_copy(w_hbm.at[pl.ds(base, GPT)], w_v)
    plsc.subcore_barrier()

    iota = lax.iota(jnp.int32, 16)

    def gather_start(j, b):
        pltpu.async_copy(feat_sh.at[src_v.at[j]], rows[b], gsem[b])

    def gather_wait(j, b):
        pltpu.make_async_copy(feat_sh.at[src_v.at[j]], rows[b],
                              gsem[b]).wait()

    def scatter_start(j, b):
        pltpu.async_copy(rows[b], acc_sh.at[dst_v.at[j]], ssem[b], add=True)

    def scatter_wait(j, b):
        pltpu.make_async_copy(rows[b], acc_sh.at[dst_v.at[j]],
                              ssem[b]).wait()

    def scale(j, b):
        rv = rows[b]
        jj = jnp.full((16,), j, dtype=jnp.int32)

        @plsc.parallel_loop(0, GROUP, 1, unroll=8)
        def _(i):
            ii = jnp.full((16,), i, dtype=jnp.int32)
            wsplat = plsc.load_gather(w_v, [jj, ii])
            row = plsc.load_gather(rv, [ii, iota])
            plsc.store_scatter(rv, [ii, iota], row * wsplat)

    gather_start(0, 0)
    gather_start(1, 1)

    def handle_group(j4, u):
        j = NBUF * j4 + u
        b = (u + 2) % NBUF
        gather_wait(j, u)
        if u < 2:
            @pl.when(j4 > 0)
            def _():
                scatter_wait(j - 2, b)
            gather_start(j + 2, b)
        else:
            @pl.when(j + 2 < GPT)
            def _():
                scatter_wait(j - 2, b)
                gather_start(j + 2, b)
        scale(j, u)
        scatter_start(j, u)
        if do_cnt:
            pltpu.async_copy(ones_v.at[0], cnt_sh.at[dst_v.at[j]], csem,
                             add=True)

    def outer_body(j4, _):
        for u in range(NBUF):
            handle_group(j4, u)
        return 0

    lax.fori_loop(0, GPT // NBUF, outer_body, 0)
    for jt in range(GPT - NBUF, GPT):
        scatter_wait(jt, jt % NBUF)
    if do_cnt:
        def cnt_drain(j, _):
            pltpu.make_async_copy(ones_v.at[0], cnt_sh.at[dst_v.at[j]],
                                  csem).wait()
            return 0
        lax.fori_loop(0, GPT, cnt_drain, 0)
    plsc.subcore_barrier()

    pltpu.sync_copy(acc_sh.at[pl.ds(base_r, ROWS_PER_SUB)],
                    acc_out.at[c, pl.ds(base_r, ROWS_PER_SUB)])
    if do_cnt:
        @pl.when(s == 0)
        def _():
            pltpu.sync_copy(cnt_sh, cnt_out.at[pl.ds(c * N_PAD, N_PAD)])


def _sc_pass2_body(acc1_hbm, cnt_hbm, xr_hbm, b1_hbm, e2_hbm,
                   w_hbm, acc_out, h_out,
                   src_v, dst_v, w_v, a0_v, a1_v, c0_v, c1_v, xr_v, b_v, h_v,
                   rows0, rows1, rows2, rows3, acc_sh, h_sh,
                   gsem0, gsem1, gsem2, gsem3, ssem0, ssem1, ssem2, ssem3):
    c = lax.axis_index("c")
    s = lax.axis_index("s")
    wid = c * NS + s
    rows = (rows0, rows1, rows2, rows3)
    gsem = (gsem0, gsem1, gsem2, gsem3)
    ssem = (ssem0, ssem1, ssem2, ssem3)
    iota = lax.iota(jnp.int32, 16)
    base_r = s * ROWS_PER_SUB

    _zero_slab_via(rows0, acc_sh, base_r)
    pltpu.sync_copy(acc1_hbm.at[0, pl.ds(base_r, ROWS_PER_SUB)], a0_v)
    pltpu.sync_copy(acc1_hbm.at[1, pl.ds(base_r, ROWS_PER_SUB)], a1_v)
    pltpu.sync_copy(cnt_hbm.at[pl.ds(base_r, ROWS_PER_SUB)], c0_v)
    pltpu.sync_copy(cnt_hbm.at[pl.ds(N_PAD + base_r, ROWS_PER_SUB)], c1_v)
    pltpu.sync_copy(xr_hbm.at[pl.ds(base_r, ROWS_PER_SUB)], xr_v)
    pltpu.sync_copy(b1_hbm, b_v)
    bias = b_v[...]

    @plsc.parallel_loop(0, ROWS_PER_SUB, 1, unroll=4)
    def _(r):
        rr = jnp.full((16,), r, dtype=jnp.int32)
        a = plsc.load_gather(a0_v, [rr, iota]) + plsc.load_gather(a1_v, [rr, iota])
        cs = plsc.load_gather(c0_v, [rr]) + plsc.load_gather(c1_v, [rr])
        mean = a / jnp.maximum(cs, 1.0)
        hrow = jnp.maximum(mean + plsc.load_gather(xr_v, [rr, iota]) + bias,
                           0.0)
        plsc.store_scatter(h_v, [rr, iota], hrow)

    pltpu.sync_copy(h_v, h_sh.at[pl.ds(base_r, ROWS_PER_SUB)])

    @pl.when(c == 0)
    def _():
        pltpu.sync_copy(h_v, h_out.at[pl.ds(base_r, ROWS_PER_SUB)])

    base = wid * GPT
    pltpu.sync_copy(e2_hbm.at[0, pl.ds(base, GPT)], src_v)
    pltpu.sync_copy(e2_hbm.at[1, pl.ds(base, GPT)], dst_v)
    pltpu.sync_copy(w_hbm.at[pl.ds(base, GPT)], w_v)
    plsc.subcore_barrier()

    def gather_start(j, b):
        pltpu.async_copy(h_sh.at[src_v.at[j]], rows[b], gsem[b])

    def gather_wait(j, b):
        pltpu.make_async_copy(h_sh.at[src_v.at[j]], rows[b], gsem[b]).wait()

    def scatter_start(j, b):
        pltpu.async_copy(rows[b], acc_sh.at[dst_v.at[j]], ssem[b], add=True)

    def scatter_wait(j, b):
        pltpu.make_async_copy(rows[b], acc_sh.at[dst_v.at[j]],
                              ssem[b]).wait()

    def scale(j, b):
        rv = rows[b]
        jj = jnp.full((16,), j, dtype=jnp.int32)

        @plsc.parallel_loop(0, GROUP, 1, unroll=8)
        def _(i):
            ii = jnp.full((16,), i, dtype=jnp.int32)
            wsplat = plsc.load_gather(w_v, [jj, ii])
            row = plsc.load_gather(rv, [ii, iota])
            plsc.store_scatter(rv, [ii, iota], row * wsplat)

    gather_start(0, 0)
    gather_start(1, 1)

    def handle_group(j4, u):
        j = NBUF * j4 + u
        b = (u + 2) % NBUF
        gather_wait(j, u)
        if u < 2:
            @pl.when(j4 > 0)
            def _():
                scatter_wait(j - 2, b)
            gather_start(j + 2, b)
        else:
            @pl.when(j + 2 < GPT)
            def _():
                scatter_wait(j - 2, b)
                gather_start(j + 2, b)
        scale(j, u)
        scatter_start(j, u)

    def outer_body(j4, _):
        for u in range(NBUF):
            handle_group(j4, u)
        return 0

    lax.fori_loop(0, GPT // NBUF, outer_body, 0)
    for jt in range(GPT - NBUF, GPT):
        scatter_wait(jt, jt % NBUF)
    plsc.subcore_barrier()

    pltpu.sync_copy(acc_sh.at[pl.ds(base_r, ROWS_PER_SUB)],
                    acc_out.at[c, pl.ds(base_r, ROWS_PER_SUB)])


def _make_sc_pass2():
    mesh = plsc.VectorSubcoreMesh(core_axis_name="c", subcore_axis_name="s",
                                  num_cores=NC, num_subcores=NS)
    out_type = (jax.ShapeDtypeStruct((NC, N_PAD, D_HID), jnp.float32),
                jax.ShapeDtypeStruct((N_PAD, D_HID), jnp.float32))
    scratch = (
        pltpu.VMEM((GPT, GROUP), jnp.int32),
        pltpu.VMEM((GPT, GROUP), jnp.int32),
        pltpu.VMEM((GPT, GROUP), jnp.float32),
        pltpu.VMEM((ROWS_PER_SUB, D_HID), jnp.float32),
        pltpu.VMEM((ROWS_PER_SUB, D_HID), jnp.float32),
        pltpu.VMEM((ROWS_PER_SUB,), jnp.float32),
        pltpu.VMEM((ROWS_PER_SUB,), jnp.float32),
        pltpu.VMEM((ROWS_PER_SUB, D_HID), jnp.float32),
        pltpu.VMEM((D_HID,), jnp.float32),
        pltpu.VMEM((ROWS_PER_SUB, D_HID), jnp.float32),
        pltpu.VMEM((GROUP, D_HID), jnp.float32),
        pltpu.VMEM((GROUP, D_HID), jnp.float32),
        pltpu.VMEM((GROUP, D_HID), jnp.float32),
        pltpu.VMEM((GROUP, D_HID), jnp.float32),
        pltpu.VMEM_SHARED((N_PAD, D_HID), jnp.float32),
        pltpu.VMEM_SHARED((N_PAD, D_HID), jnp.float32),
    ) + (pltpu.SemaphoreType.DMA,) * 8

    return pl.kernel(_sc_pass2_body, out_type=out_type, mesh=mesh,
                     scratch_types=scratch,
                     compiler_params=pltpu.CompilerParams(
                         needs_layout_passes=False,
                         use_tc_tiling_on_sc=False))


def _make_sc_pass(do_cnt):
    mesh = plsc.VectorSubcoreMesh(core_axis_name="c", subcore_axis_name="s",
                                  num_cores=NC, num_subcores=NS)
    out_type = (jax.ShapeDtypeStruct((NC, N_PAD, D_HID), jnp.float32),
                jax.ShapeDtypeStruct((NC * N_PAD,), jnp.float32))
    scratch = (
        pltpu.VMEM((GPT, GROUP), jnp.int32),
        pltpu.VMEM((GPT, GROUP), jnp.int32),
        pltpu.VMEM((GPT, GROUP), jnp.float32),
        pltpu.VMEM((2, GROUP), jnp.float32),
        pltpu.VMEM((GROUP, D_HID), jnp.float32),
        pltpu.VMEM((GROUP, D_HID), jnp.float32),
        pltpu.VMEM((GROUP, D_HID), jnp.float32),
        pltpu.VMEM((GROUP, D_HID), jnp.float32),
        pltpu.VMEM_SHARED((N_PAD, D_HID), jnp.float32),
        pltpu.VMEM_SHARED((N_PAD,), jnp.float32),
        pltpu.VMEM_SHARED((N_PAD, D_HID), jnp.float32),
    ) + (pltpu.SemaphoreType.DMA,) * 9

    def body(feat, e2, w, acc_out, cnt_out,
             src_v, dst_v, w_v, ones_v, rows0, rows1, rows2, rows3,
             acc_sh, cnt_sh, feat_sh, gsem0, gsem1, gsem2, gsem3,
             ssem0, ssem1, ssem2, ssem3, csem):
        _sc_pass(feat, e2, w, acc_out, cnt_out,
                 src_v, dst_v, w_v, ones_v, rows0, rows1, rows2, rows3,
                 acc_sh, cnt_sh, feat_sh, gsem0, gsem1, gsem2, gsem3,
                 ssem0, ssem1, ssem2, ssem3, csem, do_cnt=do_cnt)

    return pl.kernel(body, out_type=out_type, mesh=mesh,
                     scratch_types=scratch,
                     compiler_params=pltpu.CompilerParams(
                         needs_layout_passes=False,
                         use_tc_tiling_on_sc=False))


_sc_pass1 = _make_sc_pass(True)
_sc_pass2 = _make_sc_pass2()


def _mm_body(x_ref, wl_ref, wr_ref, xl_ref, xr_ref):
    x = x_ref[...]
    xl_ref[...] = jnp.dot(x, wl_ref[...], preferred_element_type=jnp.float32)
    xr_ref[...] = jnp.dot(x, wr_ref[...], preferred_element_type=jnp.float32)


def _out_body(q0_ref, q1_ref, c0_ref, c1_ref, h_ref, wl_ref, wr_ref, b_ref,
              o_ref):
    cnt = jnp.maximum(c0_ref[0] + c1_ref[0], 1.0)
    mean = (q0_ref[0] + q1_ref[0]) / cnt
    o_ref[...] = (jnp.dot(mean, wl_ref[...], preferred_element_type=jnp.float32)
                  + jnp.dot(h_ref[...], wr_ref[...],
                            preferred_element_type=jnp.float32)
                  + b_ref[...])


def _row_spec(width):
    return pl.BlockSpec((M_BLK, width), lambda i: (i, 0))


def _full_spec(shape):
    return pl.BlockSpec(shape, lambda i: (0,) * len(shape))


def kernel(x, edge_index, edge_weight, W_l1, b_l1, W_r1, b_r1,
           W_l2, b_l2, W_r2, b_r2):
    f32 = jnp.float32
    pad = NE_PAD - N_EDGES
    pad2 = jnp.stack([jnp.zeros((pad,), jnp.int32),
                      jnp.full((pad,), N_PAD - 1, jnp.int32)])
    e2 = jnp.concatenate([edge_index.astype(jnp.int32), pad2], axis=1
                         ).reshape(2, NW * GPT, GROUP)
    w = jnp.concatenate([edge_weight, jnp.zeros((pad,), f32)]
                        ).reshape(NW * GPT, GROUP)

    grid = N_NODES // M_BLK
    grid_a = N_PAD // 1024

    xl, xr = pl.pallas_call(
        _mm_body,
        grid=(grid_a,),
        in_specs=[pl.BlockSpec((1024, D_IN), lambda i: (i, 0)),
                  _full_spec((D_IN, D_HID)), _full_spec((D_IN, D_HID))],
        out_specs=[pl.BlockSpec((1024, D_HID), lambda i: (i, 0))] * 2,
        out_shape=[jax.ShapeDtypeStruct((N_PAD, D_HID), f32)] * 2,
    )(x, W_l1, W_r1)

    acc1, cnt = _sc_pass1(xl, e2, w)
    b1 = b_l1 + b_r1

    acc2, h = _sc_pass2(acc1, cnt, xr, b1, e2, w)
    cnt3 = cnt.reshape(NC, N_PAD, 1)

    b2 = (b_l2 + b_r2).reshape(1, D_OUT)
    out = pl.pallas_call(
        _out_body,
        grid=(grid,),
        in_specs=[pl.BlockSpec((1, M_BLK, D_HID), lambda i: (0, i, 0)),
                  pl.BlockSpec((1, M_BLK, D_HID), lambda i: (1, i, 0)),
                  pl.BlockSpec((1, M_BLK, 1), lambda i: (0, i, 0)),
                  pl.BlockSpec((1, M_BLK, 1), lambda i: (1, i, 0)),
                  _row_spec(D_HID), _full_spec((D_HID, D_OUT)),
                  _full_spec((D_HID, D_OUT)), _full_spec((1, D_OUT))],
        out_specs=_row_spec(D_OUT),
        out_shape=jax.ShapeDtypeStruct((N_NODES, D_OUT), f32),
    )(acc2, acc2, cnt3, cnt3, h, W_l2, W_r2, b2)

    return out


# speedup vs baseline: 26.3146x; 1.0136x over previous
import functools

import jax
import jax.numpy as jnp
from jax import lax
from jax.experimental import pallas as pl
from jax.experimental.pallas import tpu as pltpu
from jax.experimental.pallas import tpu_sc as plsc

N_NODES = 10000
N_EDGES = 320000
D_IN = 128
D_HID = 16
D_OUT = 128

NC = 2
NS = 16
NW = NC * NS
GROUP = 128
GPT = 80
NE_PAD = NW * GPT * GROUP
N_PAD = 10240
ROWS_PER_SUB = N_PAD // NS
M_BLK = 1000


NBUF = 4


def _zero_slab_via(buf, target_sh, base_r):
    for k in range(GROUP):
        buf[k, :] = jnp.zeros((D_HID,), jnp.float32)
    for k in range(ROWS_PER_SUB // GROUP):
        pltpu.sync_copy(buf, target_sh.at[pl.ds(base_r + k * GROUP, GROUP)])


def _sc_pass(feat_hbm, e2_hbm, w_hbm, acc_out, cnt_out,
             src_v, dst_v, w_v, ones_v,
             rows0, rows1, rows2, rows3, acc_sh, cnt_sh, feat_sh,
             gsem0, gsem1, gsem2, gsem3, ssem0, ssem1, ssem2, ssem3, csem,
             *, do_cnt):
    c = lax.axis_index("c")
    s = lax.axis_index("s")
    wid = c * NS + s
    rows = (rows0, rows1, rows2, rows3)
    gsem = (gsem0, gsem1, gsem2, gsem3)
    ssem = (ssem0, ssem1, ssem2, ssem3)
    base_r = s * ROWS_PER_SUB

    _zero_slab_via(rows0, acc_sh, base_r)
    pltpu.sync_copy(feat_hbm.at[pl.ds(base_r, ROWS_PER_SUB)],
                    feat_sh.at[pl.ds(base_r, ROWS_PER_SUB)])
    if do_cnt:
        for k in range(GROUP // 16):
            ones_v[0, pl.ds(k * 16, 16)] = jnp.ones((16,), jnp.float32)
            ones_v[1, pl.ds(k * 16, 16)] = jnp.zeros((16,), jnp.float32)
        for k in range(ROWS_PER_SUB // GROUP):
            pltpu.sync_copy(ones_v.at[1],
                            cnt_sh.at[pl.ds(base_r + k * GROUP, GROUP)])

    base = wid * GPT
    pltpu.sync_copy(e2_hbm.at[0, pl.ds(base, GPT)], src_v)
    pltpu.sync_copy(e2_hbm.at[1, pl.ds(base, GPT)], dst_v)
    pltpu.sync_copy(w_hbm.at[pl.ds(base, GPT)], w_v)
    plsc.subcore_barrier()

    iota = lax.iota(jnp.int32, 16)

    def gather_start(j, b):
        pltpu.async_copy(feat_sh.at[src_v.at[j]], rows[b], gsem[b])

    def gather_wait(j, b):
        pltpu.make_async_copy(feat_sh.at[src_v.at[j]], rows[b],
                              gsem[b]).wait()

    def scatter_start(j, b):
        pltpu.async_copy(rows[b], acc_sh.at[dst_v.at[j]], ssem[b], add=True)

    def scatter_wait(j, b):
        pltpu.make_async_copy(rows[b], acc_sh.at[dst_v.at[j]],
                              ssem[b]).wait()

    def scale(j, b):
        rv = rows[b]
        jj = jnp.full((16,), j, dtype=jnp.int32)

        @plsc.parallel_loop(0, GROUP, 1, unroll=8)
        def _(i):
            ii = jnp.full((16,), i, dtype=jnp.int32)
            wsplat = plsc.load_gather(w_v, [jj, ii])
            row = plsc.load_gather(rv, [ii, iota])
            plsc.store_scatter(rv, [ii, iota], row * wsplat)

    gather_start(0, 0)
    gather_start(1, 1)

    def handle_group(j4, u):
        j = NBUF * j4 + u
        b = (u + 2) % NBUF
        gather_wait(j, u)
        if u < 2:
            @pl.when(j4 > 0)
            def _():
                scatter_wait(j - 2, b)
            gather_start(j + 2, b)
        else:
            @pl.when(j + 2 < GPT)
            def _():
                scatter_wait(j - 2, b)
                gather_start(j + 2, b)
        scale(j, u)
        scatter_start(j, u)
        if do_cnt:
            pltpu.async_copy(ones_v.at[0], cnt_sh.at[dst_v.at[j]], csem,
                             add=True)

    def outer_body(j4, _):
        for u in range(NBUF):
            handle_group(j4, u)
        return 0

    lax.fori_loop(0, GPT // NBUF, outer_body, 0)
    for jt in range(GPT - NBUF, GPT):
        scatter_wait(jt, jt % NBUF)
    if do_cnt:
        def cnt_drain(j, _):
            pltpu.make_async_copy(ones_v.at[0], cnt_sh.at[dst_v.at[j]],
                                  csem).wait()
            return 0
        lax.fori_loop(0, GPT, cnt_drain, 0)
    plsc.subcore_barrier()

    pltpu.sync_copy(acc_sh.at[pl.ds(base_r, ROWS_PER_SUB)],
                    acc_out.at[c, pl.ds(base_r, ROWS_PER_SUB)])
    if do_cnt:
        @pl.when(s == 0)
        def _():
            pltpu.sync_copy(cnt_sh, cnt_out.at[pl.ds(c * N_PAD, N_PAD)])


def _sc_pass2_body(acc1_hbm, cnt_hbm, xr_hbm, b1_hbm, e2_hbm,
                   w_hbm, acc_out, h_out,
                   src_v, dst_v, w_v, a0_v, a1_v, c0_v, c1_v, xr_v, b_v, h_v,
                   rows0, rows1, rows2, rows3, acc_sh, h_sh,
                   gsem0, gsem1, gsem2, gsem3, ssem0, ssem1, ssem2, ssem3):
    c = lax.axis_index("c")
    s = lax.axis_index("s")
    wid = c * NS + s
    rows = (rows0, rows1, rows2, rows3)
    gsem = (gsem0, gsem1, gsem2, gsem3)
    ssem = (ssem0, ssem1, ssem2, ssem3)
    iota = lax.iota(jnp.int32, 16)
    base_r = s * ROWS_PER_SUB

    _zero_slab_via(rows0, acc_sh, base_r)
    pltpu.sync_copy(acc1_hbm.at[0, pl.ds(base_r, ROWS_PER_SUB)], a0_v)
    pltpu.sync_copy(acc1_hbm.at[1, pl.ds(base_r, ROWS_PER_SUB)], a1_v)
    pltpu.sync_copy(cnt_hbm.at[pl.ds(base_r, ROWS_PER_SUB)], c0_v)
    pltpu.sync_copy(cnt_hbm.at[pl.ds(N_PAD + base_r, ROWS_PER_SUB)], c1_v)
    pltpu.sync_copy(xr_hbm.at[pl.ds(base_r, ROWS_PER_SUB)], xr_v)
    pltpu.sync_copy(b1_hbm, b_v)
    bias = b_v[...]

    @plsc.parallel_loop(0, ROWS_PER_SUB, 1, unroll=4)
    def _(r):
        rr = jnp.full((16,), r, dtype=jnp.int32)
        a = plsc.load_gather(a0_v, [rr, iota]) + plsc.load_gather(a1_v, [rr, iota])
        cs = plsc.load_gather(c0_v, [rr]) + plsc.load_gather(c1_v, [rr])
        mean = a / jnp.maximum(cs, 1.0)
        hrow = jnp.maximum(mean + plsc.load_gather(xr_v, [rr, iota]) + bias,
                           0.0)
        plsc.store_scatter(h_v, [rr, iota], hrow)

    pltpu.sync_copy(h_v, h_sh.at[pl.ds(base_r, ROWS_PER_SUB)])

    @pl.when(c == 0)
    def _():
        pltpu.sync_copy(h_v, h_out.at[pl.ds(base_r, ROWS_PER_SUB)])

    base = wid * GPT
    pltpu.sync_copy(e2_hbm.at[0, pl.ds(base, GPT)], src_v)
    pltpu.sync_copy(e2_hbm.at[1, pl.ds(base, GPT)], dst_v)
    pltpu.sync_copy(w_hbm.at[pl.ds(base, GPT)], w_v)
    plsc.subcore_barrier()

    def gather_start(j, b):
        pltpu.async_copy(h_sh.at[src_v.at[j]], rows[b], gsem[b])

    def gather_wait(j, b):
        pltpu.make_async_copy(h_sh.at[src_v.at[j]], rows[b], gsem[b]).wait()

    def scatter_start(j, b):
        pltpu.async_copy(rows[b], acc_sh.at[dst_v.at[j]], ssem[b], add=True)

    def scatter_wait(j, b):
        pltpu.make_async_copy(rows[b], acc_sh.at[dst_v.at[j]],
                              ssem[b]).wait()

    def scale(j, b):
        rv = rows[b]
        jj = jnp.full((16,), j, dtype=jnp.int32)

        @plsc.parallel_loop(0, GROUP, 1, unroll=8)
        def _(i):
            ii = jnp.full((16,), i, dtype=jnp.int32)
            wsplat = plsc.load_gather(w_v, [jj, ii])
            row = plsc.load_gather(rv, [ii, iota])
            plsc.store_scatter(rv, [ii, iota], row * wsplat)

    gather_start(0, 0)
    gather_start(1, 1)

    def handle_group(j4, u):
        j = NBUF * j4 + u
        b = (u + 2) % NBUF
        gather_wait(j, u)
        if u < 2:
            @pl.when(j4 > 0)
            def _():
                scatter_wait(j - 2, b)
            gather_start(j + 2, b)
        else:
            @pl.when(j + 2 < GPT)
            def _():
                scatter_wait(j - 2, b)
                gather_start(j + 2, b)
        scale(j, u)
        scatter_start(j, u)

    def outer_body(j4, _):
        for u in range(NBUF):
            handle_group(j4, u)
        return 0

    lax.fori_loop(0, GPT // NBUF, outer_body, 0)
    for jt in range(GPT - NBUF, GPT):
        scatter_wait(jt, jt % NBUF)
    plsc.subcore_barrier()

    pltpu.sync_copy(acc_sh.at[pl.ds(base_r, ROWS_PER_SUB)], a0_v)

    @plsc.parallel_loop(0, ROWS_PER_SUB, 1, unroll=4)
    def _(r):
        rr = jnp.full((16,), r, dtype=jnp.int32)
        cs = plsc.load_gather(c0_v, [rr]) + plsc.load_gather(c1_v, [rr])
        q = plsc.load_gather(a0_v, [rr, iota]) / jnp.maximum(cs, 1.0)
        plsc.store_scatter(a0_v, [rr, iota], q)

    pltpu.sync_copy(a0_v, acc_out.at[c, pl.ds(base_r, ROWS_PER_SUB)])


def _make_sc_pass2():
    mesh = plsc.VectorSubcoreMesh(core_axis_name="c", subcore_axis_name="s",
                                  num_cores=NC, num_subcores=NS)
    out_type = (jax.ShapeDtypeStruct((NC, N_PAD, D_HID), jnp.float32),
                jax.ShapeDtypeStruct((N_PAD, D_HID), jnp.float32))
    scratch = (
        pltpu.VMEM((GPT, GROUP), jnp.int32),
        pltpu.VMEM((GPT, GROUP), jnp.int32),
        pltpu.VMEM((GPT, GROUP), jnp.float32),
        pltpu.VMEM((ROWS_PER_SUB, D_HID), jnp.float32),
        pltpu.VMEM((ROWS_PER_SUB, D_HID), jnp.float32),
        pltpu.VMEM((ROWS_PER_SUB,), jnp.float32),
        pltpu.VMEM((ROWS_PER_SUB,), jnp.float32),
        pltpu.VMEM((ROWS_PER_SUB, D_HID), jnp.float32),
        pltpu.VMEM((D_HID,), jnp.float32),
        pltpu.VMEM((ROWS_PER_SUB, D_HID), jnp.float32),
        pltpu.VMEM((GROUP, D_HID), jnp.float32),
        pltpu.VMEM((GROUP, D_HID), jnp.float32),
        pltpu.VMEM((GROUP, D_HID), jnp.float32),
        pltpu.VMEM((GROUP, D_HID), jnp.float32),
        pltpu.VMEM_SHARED((N_PAD, D_HID), jnp.float32),
        pltpu.VMEM_SHARED((N_PAD, D_HID), jnp.float32),
    ) + (pltpu.SemaphoreType.DMA,) * 8

    return pl.kernel(_sc_pass2_body, out_type=out_type, mesh=mesh,
                     scratch_types=scratch,
                     compiler_params=pltpu.CompilerParams(
                         needs_layout_passes=False,
                         use_tc_tiling_on_sc=False))


def _make_sc_pass(do_cnt):
    mesh = plsc.VectorSubcoreMesh(core_axis_name="c", subcore_axis_name="s",
                                  num_cores=NC, num_subcores=NS)
    out_type = (jax.ShapeDtypeStruct((NC, N_PAD, D_HID), jnp.float32),
                jax.ShapeDtypeStruct((NC * N_PAD,), jnp.float32))
    scratch = (
        pltpu.VMEM((GPT, GROUP), jnp.int32),
        pltpu.VMEM((GPT, GROUP), jnp.int32),
        pltpu.VMEM((GPT, GROUP), jnp.float32),
        pltpu.VMEM((2, GROUP), jnp.float32),
        pltpu.VMEM((GROUP, D_HID), jnp.float32),
        pltpu.VMEM((GROUP, D_HID), jnp.float32),
        pltpu.VMEM((GROUP, D_HID), jnp.float32),
        pltpu.VMEM((GROUP, D_HID), jnp.float32),
        pltpu.VMEM_SHARED((N_PAD, D_HID), jnp.float32),
        pltpu.VMEM_SHARED((N_PAD,), jnp.float32),
        pltpu.VMEM_SHARED((N_PAD, D_HID), jnp.float32),
    ) + (pltpu.SemaphoreType.DMA,) * 9

    def body(feat, e2, w, acc_out, cnt_out,
             src_v, dst_v, w_v, ones_v, rows0, rows1, rows2, rows3,
             acc_sh, cnt_sh, feat_sh, gsem0, gsem1, gsem2, gsem3,
             ssem0, ssem1, ssem2, ssem3, csem):
        _sc_pass(feat, e2, w, acc_out, cnt_out,
                 src_v, dst_v, w_v, ones_v, rows0, rows1, rows2, rows3,
                 acc_sh, cnt_sh, feat_sh, gsem0, gsem1, gsem2, gsem3,
                 ssem0, ssem1, ssem2, ssem3, csem, do_cnt=do_cnt)

    return pl.kernel(body, out_type=out_type, mesh=mesh,
                     scratch_types=scratch,
                     compiler_params=pltpu.CompilerParams(
                         needs_layout_passes=False,
                         use_tc_tiling_on_sc=False))


_sc_pass1 = _make_sc_pass(True)
_sc_pass2 = _make_sc_pass2()


def _mm_body(x_ref, wl_ref, wr_ref, xl_ref, xr_ref):
    x = x_ref[...]
    xl_ref[...] = jnp.dot(x, wl_ref[...], preferred_element_type=jnp.float32)
    xr_ref[...] = jnp.dot(x, wr_ref[...], preferred_element_type=jnp.float32)


def _out_body(q0_ref, q1_ref, h_ref, wl_ref, wr_ref, b_ref, o_ref):
    mean = q0_ref[0] + q1_ref[0]
    o_ref[...] = (jnp.dot(mean, wl_ref[...], preferred_element_type=jnp.float32)
                  + jnp.dot(h_ref[...], wr_ref[...],
                            preferred_element_type=jnp.float32)
                  + b_ref[...])


def _row_spec(width):
    return pl.BlockSpec((M_BLK, width), lambda i: (i, 0))


def _full_spec(shape):
    return pl.BlockSpec(shape, lambda i: (0,) * len(shape))


def kernel(x, edge_index, edge_weight, W_l1, b_l1, W_r1, b_r1,
           W_l2, b_l2, W_r2, b_r2):
    f32 = jnp.float32
    pad = NE_PAD - N_EDGES
    pad2 = jnp.stack([jnp.zeros((pad,), jnp.int32),
                      jnp.full((pad,), N_PAD - 1, jnp.int32)])
    e2 = jnp.concatenate([edge_index.astype(jnp.int32), pad2], axis=1
                         ).reshape(2, NW * GPT, GROUP)
    w = jnp.concatenate([edge_weight, jnp.zeros((pad,), f32)]
                        ).reshape(NW * GPT, GROUP)

    grid = N_NODES // M_BLK
    grid_a = N_PAD // 1024

    xl, xr = pl.pallas_call(
        _mm_body,
        grid=(grid_a,),
        in_specs=[pl.BlockSpec((1024, D_IN), lambda i: (i, 0)),
                  _full_spec((D_IN, D_HID)), _full_spec((D_IN, D_HID))],
        out_specs=[pl.BlockSpec((1024, D_HID), lambda i: (i, 0))] * 2,
        out_shape=[jax.ShapeDtypeStruct((N_PAD, D_HID), f32)] * 2,
    )(x, W_l1, W_r1)

    acc1, cnt = _sc_pass1(xl, e2, w)
    b1 = b_l1 + b_r1

    acc2, h = _sc_pass2(acc1, cnt, xr, b1, e2, w)

    b2 = (b_l2 + b_r2).reshape(1, D_OUT)
    out = pl.pallas_call(
        _out_body,
        grid=(grid,),
        in_specs=[pl.BlockSpec((1, M_BLK, D_HID), lambda i: (0, i, 0)),
                  pl.BlockSpec((1, M_BLK, D_HID), lambda i: (1, i, 0)),
                  _row_spec(D_HID), _full_spec((D_HID, D_OUT)),
                  _full_spec((D_HID, D_OUT)), _full_spec((1, D_OUT))],
        out_specs=_row_spec(D_OUT),
        out_shape=jax.ShapeDtypeStruct((N_NODES, D_OUT), f32),
    )(acc2, acc2, h, W_l2, W_r2, b2)

    return out


# speedup vs baseline: 27.3314x; 1.0386x over previous
import functools

import jax
import jax.numpy as jnp
from jax import lax
from jax.experimental import pallas as pl
from jax.experimental.pallas import tpu as pltpu
from jax.experimental.pallas import tpu_sc as plsc

N_NODES = 10000
N_EDGES = 320000
D_IN = 128
D_HID = 16
D_OUT = 128

NC = 2
NS = 16
NW = NC * NS
GROUP = 128
GPT = 80
NE_PAD = NW * GPT * GROUP
N_PAD = 10240
ROWS_PER_SUB = N_PAD // NS
M_BLK = 2000


NBUF = 4


def _zero_slab_via(buf, target_sh, base_r):
    for k in range(GROUP):
        buf[k, :] = jnp.zeros((D_HID,), jnp.float32)
    for k in range(ROWS_PER_SUB // GROUP):
        pltpu.sync_copy(buf, target_sh.at[pl.ds(base_r + k * GROUP, GROUP)])


def _sc_pass(feat_hbm, e2_hbm, w_hbm, acc_out, cnt_out,
             src_v, dst_v, w_v, ones_v,
             rows0, rows1, rows2, rows3, acc_sh, cnt_sh, feat_sh,
             gsem0, gsem1, gsem2, gsem3, ssem0, ssem1, ssem2, ssem3, csem,
             *, do_cnt):
    c = lax.axis_index("c")
    s = lax.axis_index("s")
    wid = c * NS + s
    rows = (rows0, rows1, rows2, rows3)
    gsem = (gsem0, gsem1, gsem2, gsem3)
    ssem = (ssem0, ssem1, ssem2, ssem3)
    base_r = s * ROWS_PER_SUB

    _zero_slab_via(rows0, acc_sh, base_r)
    pltpu.sync_copy(feat_hbm.at[pl.ds(base_r, ROWS_PER_SUB)],
                    feat_sh.at[pl.ds(base_r, ROWS_PER_SUB)])
    if do_cnt:
        for k in range(GROUP // 16):
            ones_v[0, pl.ds(k * 16, 16)] = jnp.ones((16,), jnp.float32)
            ones_v[1, pl.ds(k * 16, 16)] = jnp.zeros((16,), jnp.float32)
        for k in range(ROWS_PER_SUB // GROUP):
            pltpu.sync_copy(ones_v.at[1],
                            cnt_sh.at[pl.ds(base_r + k * GROUP, GROUP)])

    base = wid * GPT
    pltpu.sync_copy(e2_hbm.at[0, pl.ds(base, GPT)], src_v)
    pltpu.sync_copy(e2_hbm.at[1, pl.ds(base, GPT)], dst_v)
    pltpu.sync_copy(w_hbm.at[pl.ds(base, GPT)], w_v)
    plsc.subcore_barrier()

    iota = lax.iota(jnp.int32, 16)

    def gather_start(j, b):
        pltpu.async_copy(feat_sh.at[src_v.at[j]], rows[b], gsem[b])

    def gather_wait(j, b):
        pltpu.make_async_copy(feat_sh.at[src_v.at[j]], rows[b],
                              gsem[b]).wait()

    def scatter_start(j, b):
        pltpu.async_copy(rows[b], acc_sh.at[dst_v.at[j]], ssem[b], add=True)

    def scatter_wait(j, b):
        pltpu.make_async_copy(rows[b], acc_sh.at[dst_v.at[j]],
                              ssem[b]).wait()

    def scale(j, b):
        rv = rows[b]
        jj = jnp.full((16,), j, dtype=jnp.int32)

        @plsc.parallel_loop(0, GROUP, 1, unroll=8)
        def _(i):
            ii = jnp.full((16,), i, dtype=jnp.int32)
            wsplat = plsc.load_gather(w_v, [jj, ii])
            row = plsc.load_gather(rv, [ii, iota])
            plsc.store_scatter(rv, [ii, iota], row * wsplat)

    gather_start(0, 0)
    gather_start(1, 1)

    def handle_group(j4, u):
        j = NBUF * j4 + u
        b = (u + 2) % NBUF
        gather_wait(j, u)
        if u < 2:
            @pl.when(j4 > 0)
            def _():
                scatter_wait(j - 2, b)
            gather_start(j + 2, b)
        else:
            @pl.when(j + 2 < GPT)
            def _():
                scatter_wait(j - 2, b)
                gather_start(j + 2, b)
        scale(j, u)
        scatter_start(j, u)
        if do_cnt:
            pltpu.async_copy(ones_v.at[0], cnt_sh.at[dst_v.at[j]], csem,
                             add=True)

    def outer_body(j4, _):
        for u in range(NBUF):
            handle_group(j4, u)
        return 0

    lax.fori_loop(0, GPT // NBUF, outer_body, 0)
    for jt in range(GPT - NBUF, GPT):
        scatter_wait(jt, jt % NBUF)
    if do_cnt:
        def cnt_drain(j, _):
            pltpu.make_async_copy(ones_v.at[0], cnt_sh.at[dst_v.at[j]],
                                  csem).wait()
            return 0
        lax.fori_loop(0, GPT, cnt_drain, 0)
    plsc.subcore_barrier()

    pltpu.sync_copy(acc_sh.at[pl.ds(base_r, ROWS_PER_SUB)],
                    acc_out.at[c, pl.ds(base_r, ROWS_PER_SUB)])
    if do_cnt:
        @pl.when(s == 0)
        def _():
            pltpu.sync_copy(cnt_sh, cnt_out.at[pl.ds(c * N_PAD, N_PAD)])


def _sc_pass2_body(acc1_hbm, cnt_hbm, xr_hbm, b1_hbm, e2_hbm,
                   w_hbm, acc_out, h_out,
                   src_v, dst_v, w_v, a0_v, a1_v, c0_v, c1_v, xr_v, b_v, h_v,
                   rows0, rows1, rows2, rows3, acc_sh, h_sh,
                   gsem0, gsem1, gsem2, gsem3, ssem0, ssem1, ssem2, ssem3):
    c = lax.axis_index("c")
    s = lax.axis_index("s")
    wid = c * NS + s
    rows = (rows0, rows1, rows2, rows3)
    gsem = (gsem0, gsem1, gsem2, gsem3)
    ssem = (ssem0, ssem1, ssem2, ssem3)
    iota = lax.iota(jnp.int32, 16)
    base_r = s * ROWS_PER_SUB

    _zero_slab_via(rows0, acc_sh, base_r)
    pltpu.sync_copy(acc1_hbm.at[0, pl.ds(base_r, ROWS_PER_SUB)], a0_v)
    pltpu.sync_copy(acc1_hbm.at[1, pl.ds(base_r, ROWS_PER_SUB)], a1_v)
    pltpu.sync_copy(cnt_hbm.at[pl.ds(base_r, ROWS_PER_SUB)], c0_v)
    pltpu.sync_copy(cnt_hbm.at[pl.ds(N_PAD + base_r, ROWS_PER_SUB)], c1_v)
    pltpu.sync_copy(xr_hbm.at[pl.ds(base_r, ROWS_PER_SUB)], xr_v)
    pltpu.sync_copy(b1_hbm, b_v)
    bias = b_v[...]

    @plsc.parallel_loop(0, ROWS_PER_SUB, 1, unroll=4)
    def _(r):
        rr = jnp.full((16,), r, dtype=jnp.int32)
        a = plsc.load_gather(a0_v, [rr, iota]) + plsc.load_gather(a1_v, [rr, iota])
        cs = plsc.load_gather(c0_v, [rr]) + plsc.load_gather(c1_v, [rr])
        mean = a / jnp.maximum(cs, 1.0)
        hrow = jnp.maximum(mean + plsc.load_gather(xr_v, [rr, iota]) + bias,
                           0.0)
        plsc.store_scatter(h_v, [rr, iota], hrow)

    pltpu.sync_copy(h_v, h_sh.at[pl.ds(base_r, ROWS_PER_SUB)])

    @pl.when(c == 0)
    def _():
        pltpu.sync_copy(h_v, h_out.at[pl.ds(base_r, ROWS_PER_SUB)])

    base = wid * GPT
    pltpu.sync_copy(e2_hbm.at[0, pl.ds(base, GPT)], src_v)
    pltpu.sync_copy(e2_hbm.at[1, pl.ds(base, GPT)], dst_v)
    pltpu.sync_copy(w_hbm.at[pl.ds(base, GPT)], w_v)
    plsc.subcore_barrier()

    def gather_start(j, b):
        pltpu.async_copy(h_sh.at[src_v.at[j]], rows[b], gsem[b])

    def gather_wait(j, b):
        pltpu.make_async_copy(h_sh.at[src_v.at[j]], rows[b], gsem[b]).wait()

    def scatter_start(j, b):
        pltpu.async_copy(rows[b], acc_sh.at[dst_v.at[j]], ssem[b], add=True)

    def scatter_wait(j, b):
        pltpu.make_async_copy(rows[b], acc_sh.at[dst_v.at[j]],
                              ssem[b]).wait()

    def scale(j, b):
        rv = rows[b]
        jj = jnp.full((16,), j, dtype=jnp.int32)

        @plsc.parallel_loop(0, GROUP, 1, unroll=8)
        def _(i):
            ii = jnp.full((16,), i, dtype=jnp.int32)
            wsplat = plsc.load_gather(w_v, [jj, ii])
            row = plsc.load_gather(rv, [ii, iota])
            plsc.store_scatter(rv, [ii, iota], row * wsplat)

    gather_start(0, 0)
    gather_start(1, 1)

    def handle_group(j4, u):
        j = NBUF * j4 + u
        b = (u + 2) % NBUF
        gather_wait(j, u)
        if u < 2:
            @pl.when(j4 > 0)
            def _():
                scatter_wait(j - 2, b)
            gather_start(j + 2, b)
        else:
            @pl.when(j + 2 < GPT)
            def _():
                scatter_wait(j - 2, b)
                gather_start(j + 2, b)
        scale(j, u)
        scatter_start(j, u)

    def outer_body(j4, _):
        for u in range(NBUF):
            handle_group(j4, u)
        return 0

    lax.fori_loop(0, GPT // NBUF, outer_body, 0)
    for jt in range(GPT - NBUF, GPT):
        scatter_wait(jt, jt % NBUF)
    plsc.subcore_barrier()

    pltpu.sync_copy(acc_sh.at[pl.ds(base_r, ROWS_PER_SUB)], a0_v)

    @plsc.parallel_loop(0, ROWS_PER_SUB, 1, unroll=4)
    def _(r):
        rr = jnp.full((16,), r, dtype=jnp.int32)
        cs = plsc.load_gather(c0_v, [rr]) + plsc.load_gather(c1_v, [rr])
        q = plsc.load_gather(a0_v, [rr, iota]) / jnp.maximum(cs, 1.0)
        plsc.store_scatter(a0_v, [rr, iota], q)

    pltpu.sync_copy(a0_v, acc_out.at[c, pl.ds(base_r, ROWS_PER_SUB)])


def _make_sc_pass2():
    mesh = plsc.VectorSubcoreMesh(core_axis_name="c", subcore_axis_name="s",
                                  num_cores=NC, num_subcores=NS)
    out_type = (jax.ShapeDtypeStruct((NC, N_PAD, D_HID), jnp.float32),
                jax.ShapeDtypeStruct((N_PAD, D_HID), jnp.float32))
    scratch = (
        pltpu.VMEM((GPT, GROUP), jnp.int32),
        pltpu.VMEM((GPT, GROUP), jnp.int32),
        pltpu.VMEM((GPT, GROUP), jnp.float32),
        pltpu.VMEM((ROWS_PER_SUB, D_HID), jnp.float32),
        pltpu.VMEM((ROWS_PER_SUB, D_HID), jnp.float32),
        pltpu.VMEM((ROWS_PER_SUB,), jnp.float32),
        pltpu.VMEM((ROWS_PER_SUB,), jnp.float32),
        pltpu.VMEM((ROWS_PER_SUB, D_HID), jnp.float32),
        pltpu.VMEM((D_HID,), jnp.float32),
        pltpu.VMEM((ROWS_PER_SUB, D_HID), jnp.float32),
        pltpu.VMEM((GROUP, D_HID), jnp.float32),
        pltpu.VMEM((GROUP, D_HID), jnp.float32),
        pltpu.VMEM((GROUP, D_HID), jnp.float32),
        pltpu.VMEM((GROUP, D_HID), jnp.float32),
        pltpu.VMEM_SHARED((N_PAD, D_HID), jnp.float32),
        pltpu.VMEM_SHARED((N_PAD, D_HID), jnp.float32),
    ) + (pltpu.SemaphoreType.DMA,) * 8

    return pl.kernel(_sc_pass2_body, out_type=out_type, mesh=mesh,
                     scratch_types=scratch,
                     compiler_params=pltpu.CompilerParams(
                         needs_layout_passes=False,
                         use_tc_tiling_on_sc=False))


def _make_sc_pass(do_cnt):
    mesh = plsc.VectorSubcoreMesh(core_axis_name="c", subcore_axis_name="s",
                                  num_cores=NC, num_subcores=NS)
    out_type = (jax.ShapeDtypeStruct((NC, N_PAD, D_HID), jnp.float32),
                jax.ShapeDtypeStruct((NC * N_PAD,), jnp.float32))
    scratch = (
        pltpu.VMEM((GPT, GROUP), jnp.int32),
        pltpu.VMEM((GPT, GROUP), jnp.int32),
        pltpu.VMEM((GPT, GROUP), jnp.float32),
        pltpu.VMEM((2, GROUP), jnp.float32),
        pltpu.VMEM((GROUP, D_HID), jnp.float32),
        pltpu.VMEM((GROUP, D_HID), jnp.float32),
        pltpu.VMEM((GROUP, D_HID), jnp.float32),
        pltpu.VMEM((GROUP, D_HID), jnp.float32),
        pltpu.VMEM_SHARED((N_PAD, D_HID), jnp.float32),
        pltpu.VMEM_SHARED((N_PAD,), jnp.float32),
        pltpu.VMEM_SHARED((N_PAD, D_HID), jnp.float32),
    ) + (pltpu.SemaphoreType.DMA,) * 9

    def body(feat, e2, w, acc_out, cnt_out,
             src_v, dst_v, w_v, ones_v, rows0, rows1, rows2, rows3,
             acc_sh, cnt_sh, feat_sh, gsem0, gsem1, gsem2, gsem3,
             ssem0, ssem1, ssem2, ssem3, csem):
        _sc_pass(feat, e2, w, acc_out, cnt_out,
                 src_v, dst_v, w_v, ones_v, rows0, rows1, rows2, rows3,
                 acc_sh, cnt_sh, feat_sh, gsem0, gsem1, gsem2, gsem3,
                 ssem0, ssem1, ssem2, ssem3, csem, do_cnt=do_cnt)

    return pl.kernel(body, out_type=out_type, mesh=mesh,
                     scratch_types=scratch,
                     compiler_params=pltpu.CompilerParams(
                         needs_layout_passes=False,
                         use_tc_tiling_on_sc=False))


_sc_pass1 = _make_sc_pass(True)
_sc_pass2 = _make_sc_pass2()


def _mm_body(x_ref, wl_ref, wr_ref, xl_ref, xr_ref):
    x = x_ref[...]
    xl_ref[...] = jnp.dot(x, wl_ref[...], preferred_element_type=jnp.float32)
    xr_ref[...] = jnp.dot(x, wr_ref[...], preferred_element_type=jnp.float32)


def _out_body(q0_ref, q1_ref, h_ref, wl_ref, wr_ref, b_ref, o_ref):
    mean = q0_ref[0] + q1_ref[0]
    o_ref[...] = (jnp.dot(mean, wl_ref[...], preferred_element_type=jnp.float32)
                  + jnp.dot(h_ref[...], wr_ref[...],
                            preferred_element_type=jnp.float32)
                  + b_ref[...])


def _row_spec(width):
    return pl.BlockSpec((M_BLK, width), lambda i: (i, 0))


def _full_spec(shape):
    return pl.BlockSpec(shape, lambda i: (0,) * len(shape))


def kernel(x, edge_index, edge_weight, W_l1, b_l1, W_r1, b_r1,
           W_l2, b_l2, W_r2, b_r2):
    f32 = jnp.float32
    pad = NE_PAD - N_EDGES
    pad2 = jnp.stack([jnp.zeros((pad,), jnp.int32),
                      jnp.full((pad,), N_PAD - 1, jnp.int32)])
    e2 = jnp.concatenate([edge_index.astype(jnp.int32), pad2], axis=1
                         ).reshape(2, NW * GPT, GROUP)
    w = jnp.concatenate([edge_weight, jnp.zeros((pad,), f32)]
                        ).reshape(NW * GPT, GROUP)

    grid = N_NODES // M_BLK
    grid_a = N_PAD // 2048

    xl, xr = pl.pallas_call(
        _mm_body,
        grid=(grid_a,),
        in_specs=[pl.BlockSpec((2048, D_IN), lambda i: (i, 0)),
                  _full_spec((D_IN, D_HID)), _full_spec((D_IN, D_HID))],
        out_specs=[pl.BlockSpec((2048, D_HID), lambda i: (i, 0))] * 2,
        out_shape=[jax.ShapeDtypeStruct((N_PAD, D_HID), f32)] * 2,
    )(x, W_l1, W_r1)

    acc1, cnt = _sc_pass1(xl, e2, w)
    b1 = b_l1 + b_r1

    acc2, h = _sc_pass2(acc1, cnt, xr, b1, e2, w)

    b2 = (b_l2 + b_r2).reshape(1, D_OUT)
    out = pl.pallas_call(
        _out_body,
        grid=(grid,),
        in_specs=[pl.BlockSpec((1, M_BLK, D_HID), lambda i: (0, i, 0)),
                  pl.BlockSpec((1, M_BLK, D_HID), lambda i: (1, i, 0)),
                  _row_spec(D_HID), _full_spec((D_HID, D_OUT)),
                  _full_spec((D_HID, D_OUT)), _full_spec((1, D_OUT))],
        out_specs=_row_spec(D_OUT),
        out_shape=jax.ShapeDtypeStruct((N_NODES, D_OUT), f32),
    )(acc2, acc2, h, W_l2, W_r2, b2)

    return out


# speedup vs baseline: 29.6758x; 1.0858x over previous
import functools

import jax
import jax.numpy as jnp
from jax import lax
from jax.experimental import pallas as pl
from jax.experimental.pallas import tpu as pltpu
from jax.experimental.pallas import tpu_sc as plsc

N_NODES = 10000
N_EDGES = 320000
D_IN = 128
D_HID = 16
D_OUT = 128

NC = 2
NS = 16
NW = NC * NS
GROUP = 128
GPT = 80
NE_PAD = NW * GPT * GROUP
N_PAD = 10240
ROWS_PER_SUB = N_PAD // NS
M_BLK = 2048


NBUF = 4


def _zero_slab_via(buf, target_sh, base_r):
    for k in range(GROUP):
        buf[k, :] = jnp.zeros((D_HID,), jnp.float32)
    for k in range(ROWS_PER_SUB // GROUP):
        pltpu.sync_copy(buf, target_sh.at[pl.ds(base_r + k * GROUP, GROUP)])


def _sc_pass(feat_hbm, e2_hbm, w_hbm, acc_out, cnt_out,
             src_v, dst_v, w_v, ones_v,
             rows0, rows1, rows2, rows3, acc_sh, cnt_sh, feat_sh,
             gsem0, gsem1, gsem2, gsem3, ssem0, ssem1, ssem2, ssem3, csem,
             *, do_cnt):
    c = lax.axis_index("c")
    s = lax.axis_index("s")
    wid = c * NS + s
    rows = (rows0, rows1, rows2, rows3)
    gsem = (gsem0, gsem1, gsem2, gsem3)
    ssem = (ssem0, ssem1, ssem2, ssem3)
    base_r = s * ROWS_PER_SUB

    _zero_slab_via(rows0, acc_sh, base_r)
    pltpu.sync_copy(feat_hbm.at[pl.ds(base_r, ROWS_PER_SUB)],
                    feat_sh.at[pl.ds(base_r, ROWS_PER_SUB)])
    if do_cnt:
        for k in range(GROUP // 16):
            ones_v[0, pl.ds(k * 16, 16)] = jnp.ones((16,), jnp.float32)
            ones_v[1, pl.ds(k * 16, 16)] = jnp.zeros((16,), jnp.float32)
        for k in range(ROWS_PER_SUB // GROUP):
            pltpu.sync_copy(ones_v.at[1],
                            cnt_sh.at[pl.ds(base_r + k * GROUP, GROUP)])

    base = wid * GPT
    pltpu.sync_copy(e2_hbm.at[0, pl.ds(base, GPT)], src_v)
    pltpu.sync_copy(e2_hbm.at[1, pl.ds(base, GPT)], dst_v)
    pltpu.sync_copy(w_hbm.at[pl.ds(base, GPT)], w_v)
    plsc.subcore_barrier()

    iota = lax.iota(jnp.int32, 16)

    def gather_start(j, b):
        pltpu.async_copy(feat_sh.at[src_v.at[j]], rows[b], gsem[b])

    def gather_wait(j, b):
        pltpu.make_async_copy(feat_sh.at[src_v.at[j]], rows[b],
                              gsem[b]).wait()

    def scatter_start(j, b):
        pltpu.async_copy(rows[b], acc_sh.at[dst_v.at[j]], ssem[b], add=True)

    def scatter_wait(j, b):
        pltpu.make_async_copy(rows[b], acc_sh.at[dst_v.at[j]],
                              ssem[b]).wait()

    def scale(j, b):
        rv = rows[b]
        jj = jnp.full((16,), j, dtype=jnp.int32)

        @plsc.parallel_loop(0, GROUP, 1, unroll=8)
        def _(i):
            ii = jnp.full((16,), i, dtype=jnp.int32)
            wsplat = plsc.load_gather(w_v, [jj, ii])
            row = plsc.load_gather(rv, [ii, iota])
            plsc.store_scatter(rv, [ii, iota], row * wsplat)

    gather_start(0, 0)
    gather_start(1, 1)

    def handle_group(j4, u):
        j = NBUF * j4 + u
        b = (u + 2) % NBUF
        gather_wait(j, u)
        if u < 2:
            @pl.when(j4 > 0)
            def _():
                scatter_wait(j - 2, b)
            gather_start(j + 2, b)
        else:
            @pl.when(j + 2 < GPT)
            def _():
                scatter_wait(j - 2, b)
                gather_start(j + 2, b)
        scale(j, u)
        scatter_start(j, u)
        if do_cnt:
            pltpu.async_copy(ones_v.at[0], cnt_sh.at[dst_v.at[j]], csem,
                             add=True)

    def outer_body(j4, _):
        for u in range(NBUF):
            handle_group(j4, u)
        return 0

    lax.fori_loop(0, GPT // NBUF, outer_body, 0)
    for jt in range(GPT - NBUF, GPT):
        scatter_wait(jt, jt % NBUF)
    if do_cnt:
        def cnt_drain(j, _):
            pltpu.make_async_copy(ones_v.at[0], cnt_sh.at[dst_v.at[j]],
                                  csem).wait()
            return 0
        lax.fori_loop(0, GPT, cnt_drain, 0)
    plsc.subcore_barrier()

    pltpu.sync_copy(acc_sh.at[pl.ds(base_r, ROWS_PER_SUB)],
                    acc_out.at[c, pl.ds(base_r, ROWS_PER_SUB)])
    if do_cnt:
        @pl.when(s == 0)
        def _():
            pltpu.sync_copy(cnt_sh, cnt_out.at[pl.ds(c * N_PAD, N_PAD)])


def _sc_pass2_body(acc1_hbm, cnt_hbm, xr_hbm, b1_hbm, e2_hbm,
                   w_hbm, acc_out, h_out,
                   src_v, dst_v, w_v, a0_v, a1_v, c0_v, c1_v, xr_v, b_v,
                   h_v, h_v2,
                   rows0, rows1, rows2, rows3, acc_sh, h_sh,
                   gsem0, gsem1, gsem2, gsem3, ssem0, ssem1, ssem2, ssem3):
    c = lax.axis_index("c")
    s = lax.axis_index("s")
    wid = c * NS + s
    rows = (rows0, rows1, rows2, rows3)
    gsem = (gsem0, gsem1, gsem2, gsem3)
    ssem = (ssem0, ssem1, ssem2, ssem3)
    iota = lax.iota(jnp.int32, 16)
    base_r = s * ROWS_PER_SUB

    _zero_slab_via(rows0, acc_sh, base_r)
    pltpu.sync_copy(acc1_hbm.at[0, pl.ds(base_r, ROWS_PER_SUB)], a0_v)
    pltpu.sync_copy(acc1_hbm.at[1, pl.ds(base_r, ROWS_PER_SUB)], a1_v)
    pltpu.sync_copy(cnt_hbm.at[pl.ds(base_r, ROWS_PER_SUB)], c0_v)
    pltpu.sync_copy(cnt_hbm.at[pl.ds(N_PAD + base_r, ROWS_PER_SUB)], c1_v)
    pltpu.sync_copy(xr_hbm.at[pl.ds(base_r, ROWS_PER_SUB)], xr_v)
    pltpu.sync_copy(b1_hbm, b_v)
    bias = b_v[...]

    @plsc.parallel_loop(0, ROWS_PER_SUB, 1, unroll=4)
    def _(r):
        rr = jnp.full((16,), r, dtype=jnp.int32)
        a = plsc.load_gather(a0_v, [rr, iota]) + plsc.load_gather(a1_v, [rr, iota])
        cs = plsc.load_gather(c0_v, [rr]) + plsc.load_gather(c1_v, [rr])
        mean = a / jnp.maximum(cs, 1.0)
        hrow = jnp.maximum(mean + plsc.load_gather(xr_v, [rr, iota]) + bias,
                           0.0)
        plsc.store_scatter(h_v, [rr, iota], hrow)
        plsc.store_scatter(h_v2, [rr // 8, (rr % 8) * 16 + iota], hrow)

    pltpu.sync_copy(h_v, h_sh.at[pl.ds(base_r, ROWS_PER_SUB)])

    @pl.when(c == 0)
    def _():
        pltpu.sync_copy(h_v2, h_out.at[pl.ds(s * (ROWS_PER_SUB // 8),
                                             ROWS_PER_SUB // 8)])

    base = wid * GPT
    pltpu.sync_copy(e2_hbm.at[0, pl.ds(base, GPT)], src_v)
    pltpu.sync_copy(e2_hbm.at[1, pl.ds(base, GPT)], dst_v)
    pltpu.sync_copy(w_hbm.at[pl.ds(base, GPT)], w_v)
    plsc.subcore_barrier()

    def gather_start(j, b):
        pltpu.async_copy(h_sh.at[src_v.at[j]], rows[b], gsem[b])

    def gather_wait(j, b):
        pltpu.make_async_copy(h_sh.at[src_v.at[j]], rows[b], gsem[b]).wait()

    def scatter_start(j, b):
        pltpu.async_copy(rows[b], acc_sh.at[dst_v.at[j]], ssem[b], add=True)

    def scatter_wait(j, b):
        pltpu.make_async_copy(rows[b], acc_sh.at[dst_v.at[j]],
                              ssem[b]).wait()

    def scale(j, b):
        rv = rows[b]
        jj = jnp.full((16,), j, dtype=jnp.int32)

        @plsc.parallel_loop(0, GROUP, 1, unroll=8)
        def _(i):
            ii = jnp.full((16,), i, dtype=jnp.int32)
            wsplat = plsc.load_gather(w_v, [jj, ii])
            row = plsc.load_gather(rv, [ii, iota])
            plsc.store_scatter(rv, [ii, iota], row * wsplat)

    gather_start(0, 0)
    gather_start(1, 1)

    def handle_group(j4, u):
        j = NBUF * j4 + u
        b = (u + 2) % NBUF
        gather_wait(j, u)
        if u < 2:
            @pl.when(j4 > 0)
            def _():
                scatter_wait(j - 2, b)
            gather_start(j + 2, b)
        else:
            @pl.when(j + 2 < GPT)
            def _():
                scatter_wait(j - 2, b)
                gather_start(j + 2, b)
        scale(j, u)
        scatter_start(j, u)

    def outer_body(j4, _):
        for u in range(NBUF):
            handle_group(j4, u)
        return 0

    lax.fori_loop(0, GPT // NBUF, outer_body, 0)
    for jt in range(GPT - NBUF, GPT):
        scatter_wait(jt, jt % NBUF)
    plsc.subcore_barrier()

    pltpu.sync_copy(acc_sh.at[pl.ds(base_r, ROWS_PER_SUB)], a0_v)

    @plsc.parallel_loop(0, ROWS_PER_SUB, 1, unroll=4)
    def _(r):
        rr = jnp.full((16,), r, dtype=jnp.int32)
        cs = plsc.load_gather(c0_v, [rr]) + plsc.load_gather(c1_v, [rr])
        q = plsc.load_gather(a0_v, [rr, iota]) / jnp.maximum(cs, 1.0)
        plsc.store_scatter(h_v2, [rr // 8, (rr % 8) * 16 + iota], q)

    pltpu.sync_copy(h_v2, acc_out.at[c, pl.ds(s * (ROWS_PER_SUB // 8),
                                              ROWS_PER_SUB // 8)])


def _make_sc_pass2():
    mesh = plsc.VectorSubcoreMesh(core_axis_name="c", subcore_axis_name="s",
                                  num_cores=NC, num_subcores=NS)
    out_type = (jax.ShapeDtypeStruct((NC, N_PAD // 8, 8 * D_HID),
                                     jnp.float32),
                jax.ShapeDtypeStruct((N_PAD // 8, 8 * D_HID), jnp.float32))
    scratch = (
        pltpu.VMEM((GPT, GROUP), jnp.int32),
        pltpu.VMEM((GPT, GROUP), jnp.int32),
        pltpu.VMEM((GPT, GROUP), jnp.float32),
        pltpu.VMEM((ROWS_PER_SUB, D_HID), jnp.float32),
        pltpu.VMEM((ROWS_PER_SUB, D_HID), jnp.float32),
        pltpu.VMEM((ROWS_PER_SUB,), jnp.float32),
        pltpu.VMEM((ROWS_PER_SUB,), jnp.float32),
        pltpu.VMEM((ROWS_PER_SUB, D_HID), jnp.float32),
        pltpu.VMEM((D_HID,), jnp.float32),
        pltpu.VMEM((ROWS_PER_SUB, D_HID), jnp.float32),
        pltpu.VMEM((ROWS_PER_SUB // 8, 8 * D_HID), jnp.float32),
        pltpu.VMEM((GROUP, D_HID), jnp.float32),
        pltpu.VMEM((GROUP, D_HID), jnp.float32),
        pltpu.VMEM((GROUP, D_HID), jnp.float32),
        pltpu.VMEM((GROUP, D_HID), jnp.float32),
        pltpu.VMEM_SHARED((N_PAD, D_HID), jnp.float32),
        pltpu.VMEM_SHARED((N_PAD, D_HID), jnp.float32),
    ) + (pltpu.SemaphoreType.DMA,) * 8

    return pl.kernel(_sc_pass2_body, out_type=out_type, mesh=mesh,
                     scratch_types=scratch,
                     compiler_params=pltpu.CompilerParams(
                         needs_layout_passes=False,
                         use_tc_tiling_on_sc=False))


def _make_sc_pass(do_cnt):
    mesh = plsc.VectorSubcoreMesh(core_axis_name="c", subcore_axis_name="s",
                                  num_cores=NC, num_subcores=NS)
    out_type = (jax.ShapeDtypeStruct((NC, N_PAD, D_HID), jnp.float32),
                jax.ShapeDtypeStruct((NC * N_PAD,), jnp.float32))
    scratch = (
        pltpu.VMEM((GPT, GROUP), jnp.int32),
        pltpu.VMEM((GPT, GROUP), jnp.int32),
        pltpu.VMEM((GPT, GROUP), jnp.float32),
        pltpu.VMEM((2, GROUP), jnp.float32),
        pltpu.VMEM((GROUP, D_HID), jnp.float32),
        pltpu.VMEM((GROUP, D_HID), jnp.float32),
        pltpu.VMEM((GROUP, D_HID), jnp.float32),
        pltpu.VMEM((GROUP, D_HID), jnp.float32),
        pltpu.VMEM_SHARED((N_PAD, D_HID), jnp.float32),
        pltpu.VMEM_SHARED((N_PAD,), jnp.float32),
        pltpu.VMEM_SHARED((N_PAD, D_HID), jnp.float32),
    ) + (pltpu.SemaphoreType.DMA,) * 9

    def body(feat, e2, w, acc_out, cnt_out,
             src_v, dst_v, w_v, ones_v, rows0, rows1, rows2, rows3,
             acc_sh, cnt_sh, feat_sh, gsem0, gsem1, gsem2, gsem3,
             ssem0, ssem1, ssem2, ssem3, csem):
        _sc_pass(feat, e2, w, acc_out, cnt_out,
                 src_v, dst_v, w_v, ones_v, rows0, rows1, rows2, rows3,
                 acc_sh, cnt_sh, feat_sh, gsem0, gsem1, gsem2, gsem3,
                 ssem0, ssem1, ssem2, ssem3, csem, do_cnt=do_cnt)

    return pl.kernel(body, out_type=out_type, mesh=mesh,
                     scratch_types=scratch,
                     compiler_params=pltpu.CompilerParams(
                         needs_layout_passes=False,
                         use_tc_tiling_on_sc=False))


_sc_pass1 = _make_sc_pass(True)
_sc_pass2 = _make_sc_pass2()


def _mm_body(x_ref, wl_ref, wr_ref, xl_ref, xr_ref):
    x = x_ref[...]
    xl_ref[...] = jnp.dot(x, wl_ref[...], preferred_element_type=jnp.float32)
    xr_ref[...] = jnp.dot(x, wr_ref[...], preferred_element_type=jnp.float32)


def _out_body(q0_ref, q1_ref, h_ref, wl_ref, wr_ref, b_ref, o_ref):
    mean = q0_ref[0] + q1_ref[0]
    acc = (jnp.dot(mean, wl_ref[...], preferred_element_type=jnp.float32)
           + jnp.dot(h_ref[...], wr_ref[...],
                     preferred_element_type=jnp.float32))
    o_ref[...] = acc.reshape(M_BLK, D_OUT) + b_ref[...]


def _row_spec(width):
    return pl.BlockSpec((M_BLK, width), lambda i: (i, 0))


def _full_spec(shape):
    return pl.BlockSpec(shape, lambda i: (0,) * len(shape))


def kernel(x, edge_index, edge_weight, W_l1, b_l1, W_r1, b_r1,
           W_l2, b_l2, W_r2, b_r2):
    f32 = jnp.float32
    pad = NE_PAD - N_EDGES
    pad2 = jnp.stack([jnp.zeros((pad,), jnp.int32),
                      jnp.full((pad,), N_PAD - 1, jnp.int32)])
    e2 = jnp.concatenate([edge_index.astype(jnp.int32), pad2], axis=1
                         ).reshape(2, NW * GPT, GROUP)
    w = jnp.concatenate([edge_weight, jnp.zeros((pad,), f32)]
                        ).reshape(NW * GPT, GROUP)

    grid = (N_NODES + M_BLK - 1) // M_BLK
    grid_a = N_PAD // 2048

    xl, xr = pl.pallas_call(
        _mm_body,
        grid=(grid_a,),
        in_specs=[pl.BlockSpec((2048, D_IN), lambda i: (i, 0)),
                  _full_spec((D_IN, D_HID)), _full_spec((D_IN, D_HID))],
        out_specs=[pl.BlockSpec((2048, D_HID), lambda i: (i, 0))] * 2,
        out_shape=[jax.ShapeDtypeStruct((N_PAD, D_HID), f32)] * 2,
    )(x, W_l1, W_r1)

    acc1, cnt = _sc_pass1(xl, e2, w)
    b1 = b_l1 + b_r1

    acc2, h = _sc_pass2(acc1, cnt, xr, b1, e2, w)

    b2 = (b_l2 + b_r2).reshape(1, D_OUT)
    eye8 = jnp.eye(8, dtype=f32)
    wbd_l = jnp.kron(eye8, W_l2)
    wbd_r = jnp.kron(eye8, W_r2)
    mb8 = M_BLK // 8
    out = pl.pallas_call(
        _out_body,
        grid=(grid,),
        in_specs=[pl.BlockSpec((1, mb8, 8 * D_HID), lambda i: (0, i, 0)),
                  pl.BlockSpec((1, mb8, 8 * D_HID), lambda i: (1, i, 0)),
                  pl.BlockSpec((mb8, 8 * D_HID), lambda i: (i, 0)),
                  _full_spec((8 * D_HID, 8 * D_OUT)),
                  _full_spec((8 * D_HID, 8 * D_OUT)),
                  _full_spec((1, D_OUT))],
        out_specs=_row_spec(D_OUT),
        out_shape=jax.ShapeDtypeStruct((N_NODES, D_OUT), f32),
    )(acc2, acc2, h, wbd_l, wbd_r, b2)

    return out


# speedup vs baseline: 29.7295x; 1.0018x over previous
import jax
import jax.numpy as jnp
from jax import lax
from jax.experimental import pallas as pl
from jax.experimental.pallas import tpu as pltpu
from jax.experimental.pallas import tpu_sc as plsc

N_NODES = 10000
N_EDGES = 320000
D_IN = 128
D_HID = 16
D_OUT = 128

NC = 2
NS = 16
NW = NC * NS
GROUP = 128
GPT = 80
NE_PAD = NW * GPT * GROUP
N_PAD = 10240
ROWS_PER_SUB = N_PAD // NS
M_BLK = 2048


NBUF = 4


def _zero_slab_via(buf, target_sh, base_r):
    for k in range(GROUP):
        buf[k, :] = jnp.zeros((D_HID,), jnp.float32)
    for k in range(ROWS_PER_SUB // GROUP):
        pltpu.sync_copy(buf, target_sh.at[pl.ds(base_r + k * GROUP, GROUP)])


def _sc_pass(feat_hbm, e2_hbm, w_hbm, acc_out, cnt_out,
             src_v, dst_v, w_v, ones_v,
             rows0, rows1, rows2, rows3, acc_sh, cnt_sh, feat_sh,
             gsem0, gsem1, gsem2, gsem3, ssem0, ssem1, ssem2, ssem3, csem,
             *, do_cnt):
    c = lax.axis_index("c")
    s = lax.axis_index("s")
    wid = c * NS + s
    rows = (rows0, rows1, rows2, rows3)
    gsem = (gsem0, gsem1, gsem2, gsem3)
    ssem = (ssem0, ssem1, ssem2, ssem3)
    base_r = s * ROWS_PER_SUB

    _zero_slab_via(rows0, acc_sh, base_r)
    pltpu.sync_copy(feat_hbm.at[pl.ds(base_r, ROWS_PER_SUB)],
                    feat_sh.at[pl.ds(base_r, ROWS_PER_SUB)])
    if do_cnt:
        for k in range(GROUP // 16):
            ones_v[0, pl.ds(k * 16, 16)] = jnp.ones((16,), jnp.float32)
            ones_v[1, pl.ds(k * 16, 16)] = jnp.zeros((16,), jnp.float32)
        for k in range(ROWS_PER_SUB // GROUP):
            pltpu.sync_copy(ones_v.at[1],
                            cnt_sh.at[pl.ds(base_r + k * GROUP, GROUP)])

    base = wid * GPT
    pltpu.sync_copy(e2_hbm.at[0, pl.ds(base, GPT)], src_v)
    pltpu.sync_copy(e2_hbm.at[1, pl.ds(base, GPT)], dst_v)
    pltpu.sync_copy(w_hbm.at[pl.ds(base, GPT)], w_v)
    plsc.subcore_barrier()

    iota = lax.iota(jnp.int32, 16)

    def gather_start(j, b):
        pltpu.async_copy(feat_sh.at[src_v.at[j]], rows[b], gsem[b])

    def gather_wait(j, b):
        pltpu.make_async_copy(feat_sh.at[src_v.at[j]], rows[b],
                              gsem[b]).wait()

    def scatter_start(j, b):
        pltpu.async_copy(rows[b], acc_sh.at[dst_v.at[j]], ssem[b], add=True)

    def scatter_wait(j, b):
        pltpu.make_async_copy(rows[b], acc_sh.at[dst_v.at[j]],
                              ssem[b]).wait()

    def scale(j, b):
        rv = rows[b]
        jj = jnp.full((16,), j, dtype=jnp.int32)

        @plsc.parallel_loop(0, GROUP, 1, unroll=8)
        def _(i):
            ii = jnp.full((16,), i, dtype=jnp.int32)
            wsplat = plsc.load_gather(w_v, [jj, ii])
            row = plsc.load_gather(rv, [ii, iota])
            plsc.store_scatter(rv, [ii, iota], row * wsplat)

    gather_start(0, 0)
    gather_start(1, 1)

    def handle_group(j4, u):
        j = NBUF * j4 + u
        b = (u + 2) % NBUF
        gather_wait(j, u)
        if u < 2:
            @pl.when(j4 > 0)
            def _():
                scatter_wait(j - 2, b)
            gather_start(j + 2, b)
        else:
            @pl.when(j + 2 < GPT)
            def _():
                scatter_wait(j - 2, b)
                gather_start(j + 2, b)
        scale(j, u)
        scatter_start(j, u)
        if do_cnt:
            pltpu.async_copy(ones_v.at[0], cnt_sh.at[dst_v.at[j]], csem,
                             add=True)

    def outer_body(j4, _):
        for u in range(NBUF):
            handle_group(j4, u)
        return 0

    lax.fori_loop(0, GPT // NBUF, outer_body, 0)
    for jt in range(GPT - NBUF, GPT):
        scatter_wait(jt, jt % NBUF)
    if do_cnt:
        def cnt_drain(j, _):
            pltpu.make_async_copy(ones_v.at[0], cnt_sh.at[dst_v.at[j]],
                                  csem).wait()
            return 0
        lax.fori_loop(0, GPT, cnt_drain, 0)
    plsc.subcore_barrier()

    pltpu.sync_copy(acc_sh.at[pl.ds(base_r, ROWS_PER_SUB)],
                    acc_out.at[c, pl.ds(base_r, ROWS_PER_SUB)])
    if do_cnt:
        @pl.when(s == 0)
        def _():
            pltpu.sync_copy(cnt_sh, cnt_out.at[pl.ds(c * N_PAD, N_PAD)])


def _sc_pass2_body(acc1_hbm, cnt_hbm, xr_hbm, b1_hbm, e2_hbm,
                   w_hbm, acc_out, h_out,
                   src_v, dst_v, w_v, a0_v, a1_v, c0_v, c1_v, xr_v, b_v,
                   h_v, h_v2,
                   rows0, rows1, rows2, rows3, acc_sh, h_sh,
                   gsem0, gsem1, gsem2, gsem3, ssem0, ssem1, ssem2, ssem3):
    c = lax.axis_index("c")
    s = lax.axis_index("s")
    wid = c * NS + s
    rows = (rows0, rows1, rows2, rows3)
    gsem = (gsem0, gsem1, gsem2, gsem3)
    ssem = (ssem0, ssem1, ssem2, ssem3)
    iota = lax.iota(jnp.int32, 16)
    base_r = s * ROWS_PER_SUB

    _zero_slab_via(rows0, acc_sh, base_r)
    pltpu.sync_copy(acc1_hbm.at[0, pl.ds(base_r, ROWS_PER_SUB)], a0_v)
    pltpu.sync_copy(acc1_hbm.at[1, pl.ds(base_r, ROWS_PER_SUB)], a1_v)
    pltpu.sync_copy(cnt_hbm.at[pl.ds(base_r, ROWS_PER_SUB)], c0_v)
    pltpu.sync_copy(cnt_hbm.at[pl.ds(N_PAD + base_r, ROWS_PER_SUB)], c1_v)
    pltpu.sync_copy(xr_hbm.at[pl.ds(base_r, ROWS_PER_SUB)], xr_v)
    pltpu.sync_copy(b1_hbm, b_v)
    bias = b_v[...]

    @plsc.parallel_loop(0, ROWS_PER_SUB, 1, unroll=4)
    def _(r):
        rr = jnp.full((16,), r, dtype=jnp.int32)
        a = plsc.load_gather(a0_v, [rr, iota]) + plsc.load_gather(a1_v, [rr, iota])
        cs = plsc.load_gather(c0_v, [rr]) + plsc.load_gather(c1_v, [rr])
        mean = a / jnp.maximum(cs, 1.0)
        hrow = jnp.maximum(mean + plsc.load_gather(xr_v, [rr, iota]) + bias,
                           0.0)
        plsc.store_scatter(h_v, [rr, iota], hrow)
        plsc.store_scatter(h_v2, [rr // 8, (rr % 8) * 16 + iota], hrow)

    pltpu.sync_copy(h_v, h_sh.at[pl.ds(base_r, ROWS_PER_SUB)])

    @pl.when(c == 0)
    def _():
        pltpu.sync_copy(h_v2, h_out.at[pl.ds(s * (ROWS_PER_SUB // 8),
                                             ROWS_PER_SUB // 8)])

    base = wid * GPT
    pltpu.sync_copy(e2_hbm.at[0, pl.ds(base, GPT)], src_v)
    pltpu.sync_copy(e2_hbm.at[1, pl.ds(base, GPT)], dst_v)
    pltpu.sync_copy(w_hbm.at[pl.ds(base, GPT)], w_v)
    plsc.subcore_barrier()

    def gather_start(j, b):
        pltpu.async_copy(h_sh.at[src_v.at[j]], rows[b], gsem[b])

    def gather_wait(j, b):
        pltpu.make_async_copy(h_sh.at[src_v.at[j]], rows[b], gsem[b]).wait()

    def scatter_start(j, b):
        pltpu.async_copy(rows[b], acc_sh.at[dst_v.at[j]], ssem[b], add=True)

    def scatter_wait(j, b):
        pltpu.make_async_copy(rows[b], acc_sh.at[dst_v.at[j]],
                              ssem[b]).wait()

    def scale(j, b):
        rv = rows[b]
        jj = jnp.full((16,), j, dtype=jnp.int32)

        @plsc.parallel_loop(0, GROUP, 1, unroll=8)
        def _(i):
            ii = jnp.full((16,), i, dtype=jnp.int32)
            wsplat = plsc.load_gather(w_v, [jj, ii])
            row = plsc.load_gather(rv, [ii, iota])
            plsc.store_scatter(rv, [ii, iota], row * wsplat)

    gather_start(0, 0)
    gather_start(1, 1)

    def handle_group(j4, u):
        j = NBUF * j4 + u
        b = (u + 2) % NBUF
        gather_wait(j, u)
        if u < 2:
            @pl.when(j4 > 0)
            def _():
                scatter_wait(j - 2, b)
            gather_start(j + 2, b)
        else:
            @pl.when(j + 2 < GPT)
            def _():
                scatter_wait(j - 2, b)
                gather_start(j + 2, b)
        scale(j, u)
        scatter_start(j, u)

    def outer_body(j4, _):
        for u in range(NBUF):
            handle_group(j4, u)
        return 0

    lax.fori_loop(0, GPT // NBUF, outer_body, 0)
    for jt in range(GPT - NBUF, GPT):
        scatter_wait(jt, jt % NBUF)
    plsc.subcore_barrier()

    pltpu.sync_copy(acc_sh.at[pl.ds(base_r, ROWS_PER_SUB)], a0_v)

    @plsc.parallel_loop(0, ROWS_PER_SUB, 1, unroll=4)
    def _(r):
        rr = jnp.full((16,), r, dtype=jnp.int32)
        cs = plsc.load_gather(c0_v, [rr]) + plsc.load_gather(c1_v, [rr])
        q = plsc.load_gather(a0_v, [rr, iota]) / jnp.maximum(cs, 1.0)
        plsc.store_scatter(h_v2, [rr // 8, (rr % 8) * 16 + iota], q)

    pltpu.sync_copy(h_v2, acc_out.at[c, pl.ds(s * (ROWS_PER_SUB // 8),
                                              ROWS_PER_SUB // 8)])


def _make_sc_pass2():
    mesh = plsc.VectorSubcoreMesh(core_axis_name="c", subcore_axis_name="s",
                                  num_cores=NC, num_subcores=NS)
    out_type = (jax.ShapeDtypeStruct((NC, N_PAD // 8, 8 * D_HID),
                                     jnp.float32),
                jax.ShapeDtypeStruct((N_PAD // 8, 8 * D_HID), jnp.float32))
    scratch = (
        pltpu.VMEM((GPT, GROUP), jnp.int32),
        pltpu.VMEM((GPT, GROUP), jnp.int32),
        pltpu.VMEM((GPT, GROUP), jnp.float32),
        pltpu.VMEM((ROWS_PER_SUB, D_HID), jnp.float32),
        pltpu.VMEM((ROWS_PER_SUB, D_HID), jnp.float32),
        pltpu.VMEM((ROWS_PER_SUB,), jnp.float32),
        pltpu.VMEM((ROWS_PER_SUB,), jnp.float32),
        pltpu.VMEM((ROWS_PER_SUB, D_HID), jnp.float32),
        pltpu.VMEM((D_HID,), jnp.float32),
        pltpu.VMEM((ROWS_PER_SUB, D_HID), jnp.float32),
        pltpu.VMEM((ROWS_PER_SUB // 8, 8 * D_HID), jnp.float32),
        pltpu.VMEM((GROUP, D_HID), jnp.float32),
        pltpu.VMEM((GROUP, D_HID), jnp.float32),
        pltpu.VMEM((GROUP, D_HID), jnp.float32),
        pltpu.VMEM((GROUP, D_HID), jnp.float32),
        pltpu.VMEM_SHARED((N_PAD, D_HID), jnp.float32),
        pltpu.VMEM_SHARED((N_PAD, D_HID), jnp.float32),
    ) + (pltpu.SemaphoreType.DMA,) * 8

    return pl.kernel(_sc_pass2_body, out_type=out_type, mesh=mesh,
                     scratch_types=scratch,
                     compiler_params=pltpu.CompilerParams(
                         needs_layout_passes=False,
                         use_tc_tiling_on_sc=False))


def _make_sc_pass(do_cnt):
    mesh = plsc.VectorSubcoreMesh(core_axis_name="c", subcore_axis_name="s",
                                  num_cores=NC, num_subcores=NS)
    out_type = (jax.ShapeDtypeStruct((NC, N_PAD, D_HID), jnp.float32),
                jax.ShapeDtypeStruct((NC * N_PAD,), jnp.float32))
    scratch = (
        pltpu.VMEM((GPT, GROUP), jnp.int32),
        pltpu.VMEM((GPT, GROUP), jnp.int32),
        pltpu.VMEM((GPT, GROUP), jnp.float32),
        pltpu.VMEM((2, GROUP), jnp.float32),
        pltpu.VMEM((GROUP, D_HID), jnp.float32),
        pltpu.VMEM((GROUP, D_HID), jnp.float32),
        pltpu.VMEM((GROUP, D_HID), jnp.float32),
        pltpu.VMEM((GROUP, D_HID), jnp.float32),
        pltpu.VMEM_SHARED((N_PAD, D_HID), jnp.float32),
        pltpu.VMEM_SHARED((N_PAD,), jnp.float32),
        pltpu.VMEM_SHARED((N_PAD, D_HID), jnp.float32),
    ) + (pltpu.SemaphoreType.DMA,) * 9

    def body(feat, e2, w, acc_out, cnt_out,
             src_v, dst_v, w_v, ones_v, rows0, rows1, rows2, rows3,
             acc_sh, cnt_sh, feat_sh, gsem0, gsem1, gsem2, gsem3,
             ssem0, ssem1, ssem2, ssem3, csem):
        _sc_pass(feat, e2, w, acc_out, cnt_out,
                 src_v, dst_v, w_v, ones_v, rows0, rows1, rows2, rows3,
                 acc_sh, cnt_sh, feat_sh, gsem0, gsem1, gsem2, gsem3,
                 ssem0, ssem1, ssem2, ssem3, csem, do_cnt=do_cnt)

    return pl.kernel(body, out_type=out_type, mesh=mesh,
                     scratch_types=scratch,
                     compiler_params=pltpu.CompilerParams(
                         needs_layout_passes=False,
                         use_tc_tiling_on_sc=False))


_sc_pass1 = _make_sc_pass(True)
_sc_pass2 = _make_sc_pass2()


def _mm_body(x_ref, wl_ref, wr_ref, xl_ref, xr_ref):
    x = x_ref[...]
    xl_ref[...] = jnp.dot(x, wl_ref[...], preferred_element_type=jnp.float32)
    xr_ref[...] = jnp.dot(x, wr_ref[...], preferred_element_type=jnp.float32)


def _out_body(q0_ref, q1_ref, h_ref, wl_ref, wr_ref, b_ref, o_ref):
    mean = q0_ref[0] + q1_ref[0]
    acc = (jnp.dot(mean, wl_ref[...], preferred_element_type=jnp.float32)
           + jnp.dot(h_ref[...], wr_ref[...],
                     preferred_element_type=jnp.float32))
    o_ref[...] = acc.reshape(M_BLK, D_OUT) + b_ref[...]


def _row_spec(width):
    return pl.BlockSpec((M_BLK, width), lambda i: (i, 0))


def _full_spec(shape):
    return pl.BlockSpec(shape, lambda i: (0,) * len(shape))


def kernel(x, edge_index, edge_weight, W_l1, b_l1, W_r1, b_r1,
           W_l2, b_l2, W_r2, b_r2):
    f32 = jnp.float32
    pad = NE_PAD - N_EDGES
    pad2 = jnp.stack([jnp.zeros((pad,), jnp.int32),
                      jnp.full((pad,), N_PAD - 1, jnp.int32)])
    e2 = jnp.concatenate([edge_index.astype(jnp.int32), pad2], axis=1
                         ).reshape(2, NW * GPT, GROUP)
    w = jnp.concatenate([edge_weight, jnp.zeros((pad,), f32)]
                        ).reshape(NW * GPT, GROUP)

    grid = (N_NODES + M_BLK - 1) // M_BLK
    grid_a = N_PAD // 2048

    xl, xr = pl.pallas_call(
        _mm_body,
        grid=(grid_a,),
        in_specs=[pl.BlockSpec((2048, D_IN), lambda i: (i, 0)),
                  _full_spec((D_IN, D_HID)), _full_spec((D_IN, D_HID))],
        out_specs=[pl.BlockSpec((2048, D_HID), lambda i: (i, 0))] * 2,
        out_shape=[jax.ShapeDtypeStruct((N_PAD, D_HID), f32)] * 2,
    )(x, W_l1, W_r1)

    acc1, cnt = _sc_pass1(xl, e2, w)
    b1 = b_l1 + b_r1

    acc2, h = _sc_pass2(acc1, cnt, xr, b1, e2, w)

    b2 = (b_l2 + b_r2).reshape(1, D_OUT)
    eye8 = jnp.eye(8, dtype=f32)
    wbd_l = jnp.kron(eye8, W_l2)
    wbd_r = jnp.kron(eye8, W_r2)
    mb8 = M_BLK // 8
    out = pl.pallas_call(
        _out_body,
        grid=(grid,),
        in_specs=[pl.BlockSpec((1, mb8, 8 * D_HID), lambda i: (0, i, 0)),
                  pl.BlockSpec((1, mb8, 8 * D_HID), lambda i: (1, i, 0)),
                  pl.BlockSpec((mb8, 8 * D_HID), lambda i: (i, 0)),
                  _full_spec((8 * D_HID, 8 * D_OUT)),
                  _full_spec((8 * D_HID, 8 * D_OUT)),
                  _full_spec((1, D_OUT))],
        out_specs=_row_spec(D_OUT),
        out_shape=jax.ShapeDtypeStruct((N_NODES, D_OUT), f32),
    )(acc2, acc2, h, wbd_l, wbd_r, b2)

    return out
